# Initial kernel scaffold; baseline (speedup 1.0000x reference)
import jax
import jax.numpy as jnp
from jax import lax
from jax.experimental import pallas as pl
from jax.experimental.pallas import tpu as pltpu
from jax.experimental.pallas import tpu_sc as plsc

N_A = 10000
N_S = 10000
E = 320000
EL = 100000
D = 128
H = 128

NC = 2
NS = 16
NW = NC * NS

EPT = E // NW
K = 80
NCH = EPT // K

ELP = 100352
DPT = ELP // NW
RPT = N_S // NS

_HI = lax.Precision.HIGHEST


def _tc1_body(xa_ref, xst_ref, wsrc_ref, wdst_ref, asrc_ref, adst_ref, w1_ref,
              xs_ref, a_s_ref, a_d_ref, p_a_ref):
    xa = xa_ref[...]
    xs = jnp.dot(xa, wsrc_ref[...], precision=_HI)
    xs_ref[...] = xs
    a_s_ref[...] = jnp.dot(xs, asrc_ref[...], precision=_HI)
    v = jnp.dot(wdst_ref[...], adst_ref[...], precision=_HI)
    a_d_ref[...] = jnp.dot(xst_ref[...], v, precision=_HI)
    p_a_ref[...] = jnp.dot(xa, w1_ref[...], precision=_HI)


def _tc1(xa, xst, wsrc, wdst, asrc, adst, w1):
    return pl.pallas_call(
        _tc1_body,
        out_shape=[
            jax.ShapeDtypeStruct((N_A, H), jnp.float32),
            jax.ShapeDtypeStruct((N_A, 1), jnp.float32),
            jax.ShapeDtypeStruct((N_S, 1), jnp.float32),
            jax.ShapeDtypeStruct((N_A, 1), jnp.float32),
        ],
    )(xa, xst, wsrc, wdst, asrc, adst, w1)


def _tcr_body(dn_ref, r_ref):
    s = jnp.sum(dn_ref[...], axis=0, keepdims=True)
    r_ref[...] = 1.0 / (s + 1e-16)


def _tcr(dn):
    return pl.pallas_call(
        _tcr_body,
        out_shape=jax.ShapeDtypeStruct((1, N_S), jnp.float32),
    )(dn)


def _tc2_body(parts_ref, bconv_ref, w2_ref, bh_ref, ps_ref):
    z = jnp.maximum(parts_ref[0] + parts_ref[1] + bconv_ref[...], 0.0)
    ps_ref[...] = jnp.dot(z, w2_ref[...], precision=_HI) + bh_ref[...]


def _tc2(parts, bconv, w2, bh):
    return pl.pallas_call(
        _tc2_body,
        out_shape=jax.ShapeDtypeStruct((N_S, 1), jnp.float32),
    )(parts, bconv, w2, bh)


def _mesh():
    return plsc.VectorSubcoreMesh(
        core_axis_name="c", subcore_axis_name="s",
        num_cores=NC, num_subcores=NS)


def _sca_body(a_s_hbm, a_d_hbm, src_hbm, dst_hbm, w_hbm, dn_hbm,
              as_v, ad_v, src_v, dst_v, w_v, dn_v):
    wid = lax.axis_index("s") * NC + lax.axis_index("c")
    pltpu.sync_copy(a_s_hbm, as_v)
    pltpu.sync_copy(a_d_hbm, ad_v)
    pltpu.sync_copy(src_hbm.at[wid], src_v)
    pltpu.sync_copy(dst_hbm.at[wid], dst_v)

    zeros = jnp.zeros((16,), jnp.float32)

    def zbody(i, _):
        dn_v[pl.ds(i * 16, 16)] = zeros
        return 0
    lax.fori_loop(0, N_S // 16, zbody, 0)

    def ebody(g, _):
        sl = pl.ds(g * 16, 16)
        sv = src_v[sl]
        dv = dst_v[sl]
        e = plsc.load_gather(as_v, [sv]) + plsc.load_gather(ad_v, [dv])
        e = jnp.where(e >= 0.0, e, 0.2 * e)
        w = jnp.exp(e)
        w_v[sl] = w
        plsc.addupdate_scatter(dn_v, [dv], w)
        return 0
    lax.fori_loop(0, EPT // 16, ebody, 0)

    pltpu.sync_copy(w_v, w_hbm.at[wid])
    pltpu.sync_copy(dn_v, dn_hbm.at[wid])


def _sca(a_s, a_d, src, dst):
    return pl.kernel(
        _sca_body,
        out_type=[
            jax.ShapeDtypeStruct((NW, EPT), jnp.float32),
            jax.ShapeDtypeStruct((NW, N_S), jnp.float32),
        ],
        mesh=_mesh(),
        scratch_types=[
            pltpu.VMEM((N_A,), jnp.float32),
            pltpu.VMEM((N_S,), jnp.float32),
            pltpu.VMEM((EPT,), jnp.int32),
            pltpu.VMEM((EPT,), jnp.int32),
            pltpu.VMEM((EPT,), jnp.float32),
            pltpu.VMEM((N_S,), jnp.float32),
        ],
    )(a_s, a_d, src, dst)


def _scb_body(xs_hbm, r_hbm, src_hbm, dst_hbm, w_hbm, parts_hbm,
              r_v, src_v, dst_v, coef_v, rows_v, zbuf_v, acc, sem):
    cid = lax.axis_index("c")
    sid = lax.axis_index("s")
    wid = sid * NC + cid
    pltpu.sync_copy(r_hbm, r_v)
    pltpu.sync_copy(src_hbm.at[wid], src_v)
    pltpu.sync_copy(dst_hbm.at[wid], dst_v)
    pltpu.sync_copy(w_hbm.at[wid], coef_v)

    def cbody(c, _):
        for k in range(K // 16):
            sl = pl.ds(k * 16, 16)
            dv = dst_v[c, sl]
            coef_v[c, sl] = coef_v[c, sl] * plsc.load_gather(r_v, [dv])
        return 0
    lax.fori_loop(0, NCH, cbody, 0)

    zeros = jnp.zeros((16,), jnp.float32)

    def zbody(i, _):
        for h in range(H // 16):
            zbuf_v[i, pl.ds(h * 16, 16)] = zeros
        return 0
    lax.fori_loop(0, RPT // 5, zbody, 0)
    base = sid * RPT
    for j in range(5):
        pltpu.sync_copy(zbuf_v, acc.at[pl.ds(base + j * (RPT // 5), RPT // 5)])
    plsc.subcore_barrier()

    def mbody(c, _):
        pltpu.async_copy(xs_hbm.at[src_v.at[c]], rows_v, sem).wait()

        def rbody(i, _):
            cv = jnp.full((16,), coef_v[c, i], jnp.float32)
            for h in range(H // 16):
                sl = pl.ds(h * 16, 16)
                rows_v[i, sl] = rows_v[i, sl] * cv
            return 0
        lax.fori_loop(0, K, rbody, 0)
        pltpu.sync_copy(rows_v, acc.at[dst_v.at[c]], add=True)
        return 0
    lax.fori_loop(0, NCH, mbody, 0)

    plsc.subcore_barrier()
    pltpu.sync_copy(acc.at[pl.ds(base, RPT)],
                    parts_hbm.at[cid, pl.ds(base, RPT)])


def _scb(xs, r, src, dst, w):
    return pl.kernel(
        _scb_body,
        out_type=jax.ShapeDtypeStruct((NC, N_S, H), jnp.float32),
        mesh=_mesh(),
        scratch_types=[
            pltpu.VMEM((N_S,), jnp.float32),
            pltpu.VMEM((NCH, K), jnp.int32),
            pltpu.VMEM((NCH, K), jnp.int32),
            pltpu.VMEM((NCH, K), jnp.float32),
            pltpu.VMEM((K, H), jnp.float32),
            pltpu.VMEM((RPT // 5, H), jnp.float32),
            pltpu.VMEM_SHARED((N_S, H), jnp.float32),
            pltpu.SemaphoreType.DMA,
        ],
    )(xs, r, src, dst, w)


def _scc_body(pa_hbm, ps_hbm, row_hbm, col_hbm, out_hbm,
              pa_v, ps_v, row_v, col_v, o_v):
    wid = lax.axis_index("s") * NC + lax.axis_index("c")
    pltpu.sync_copy(pa_hbm, pa_v)
    pltpu.sync_copy(ps_hbm, ps_v)
    pltpu.sync_copy(row_hbm.at[wid], row_v)
    pltpu.sync_copy(col_hbm.at[wid], col_v)

    def gbody(g, _):
        sl = pl.ds(g * 16, 16)
        rv = row_v[sl]
        cv = col_v[sl]
        logit = plsc.load_gather(pa_v, [rv]) + plsc.load_gather(ps_v, [cv])
        o_v[sl] = 1.0 / (1.0 + jnp.exp(-logit))
        return 0
    lax.fori_loop(0, DPT // 16, gbody, 0)

    pltpu.sync_copy(o_v, out_hbm.at[wid])


def _scc(pa, ps, row, col):
    return pl.kernel(
        _scc_body,
        out_type=jax.ShapeDtypeStruct((NW, DPT), jnp.float32),
        mesh=_mesh(),
        scratch_types=[
            pltpu.VMEM((N_A,), jnp.float32),
            pltpu.VMEM((N_S,), jnp.float32),
            pltpu.VMEM((DPT,), jnp.int32),
            pltpu.VMEM((DPT,), jnp.int32),
            pltpu.VMEM((DPT,), jnp.float32),
        ],
    )(pa, ps, row, col)


def kernel(x_artwork, x_style, edge_index, edge_label_index,
           W_src, W_dst, att_src, att_dst, b_conv, W_head, b_head):
    src = edge_index[0].astype(jnp.int32)
    dst = edge_index[1].astype(jnp.int32)
    row = edge_label_index[0].astype(jnp.int32)
    col = edge_label_index[1].astype(jnp.int32)

    asrc = att_src.reshape(H, 1)
    adst = att_dst.reshape(H, 1)
    w1 = W_head[:H].reshape(H, 1)
    w2 = W_head[H:].reshape(H, 1)
    bh = b_head.reshape(1, 1)

    xs, a_s, a_d, p_a = _tc1(x_artwork, x_style, W_src, W_dst, asrc, adst, w1)

    src2 = src.reshape(NW, EPT)
    dst2 = dst.reshape(NW, EPT)
    w_e, dn = _sca(a_s.reshape(N_A), a_d.reshape(N_S), src2, dst2)

    r = _tcr(dn)

    parts = _scb(xs, r.reshape(N_S),
                 src.reshape(NW, NCH, K), dst.reshape(NW, NCH, K),
                 w_e.reshape(NW, NCH, K))

    ps = _tc2(parts, b_conv, w2, bh)

    pad = jnp.zeros((ELP - EL,), jnp.int32)
    rowp = jnp.concatenate([row, pad]).reshape(NW, DPT)
    colp = jnp.concatenate([col, pad]).reshape(NW, DPT)
    out = _scc(p_a.reshape(N_A), ps.reshape(N_S), rowp, colp)

    return out.reshape(ELP)[:EL].reshape(EL, 1)


# speedup vs baseline: 15.6900x; 15.6900x over previous
import jax
import jax.numpy as jnp
from jax import lax
from jax.experimental import pallas as pl
from jax.experimental.pallas import tpu as pltpu
from jax.experimental.pallas import tpu_sc as plsc

N_A = 10000
N_S = 10000
E = 320000
EL = 100000
D = 128
H = 128

NC = 2
NS = 16
NW = NC * NS

EPT = E // NW
EPC = E // NS
K = 80
NCH = EPT // K
NCH2 = EPC // K
SCH = 10
ICH = NCH2 // SCH
SC_E = EPC // SCH
HH = H // 2

ELP = 100352
DPT = ELP // NW
N_SP = 10240
RPT = N_SP // NS
ZR = 32

_HI = lax.Precision.HIGHEST


def _tc1_body(xa_ref, xst_ref, wsrc_ref, wdst_ref, asrc_ref, adst_ref, w1_ref,
              xs_ref, a_s_ref, a_d_ref, p_a_ref):
    xa = xa_ref[...]
    xs = jnp.dot(xa, wsrc_ref[...], precision=_HI)
    xs_ref[...] = xs
    a_s_ref[...] = jnp.sum(xs * asrc_ref[...][None, :], axis=1)
    v = jnp.sum(wdst_ref[...] * adst_ref[...][None, :], axis=1)
    a_d_ref[...] = jnp.sum(xst_ref[...] * v[None, :], axis=1)
    p_a_ref[...] = jnp.sum(xa * w1_ref[...][None, :], axis=1)


def _tc1(xa, xst, wsrc, wdst, asrc, adst, w1):
    return pl.pallas_call(
        _tc1_body,
        out_shape=[
            jax.ShapeDtypeStruct((N_A, H), jnp.float32),
            jax.ShapeDtypeStruct((N_A,), jnp.float32),
            jax.ShapeDtypeStruct((N_S,), jnp.float32),
            jax.ShapeDtypeStruct((N_A,), jnp.float32),
        ],
    )(xa, xst, wsrc, wdst, asrc, adst, w1)


def _tcr_body(dn_ref, r_ref):
    s = jnp.sum(dn_ref[...], axis=0, keepdims=True)
    r_ref[...] = 1.0 / (s + 1e-16)


def _tcr(dn):
    return pl.pallas_call(
        _tcr_body,
        out_shape=jax.ShapeDtypeStruct((1, N_S), jnp.float32),
    )(dn)


def _tc2_body(parts_ref, bconv_ref, w2_ref, bh_ref, ps_ref):
    z = jnp.maximum(parts_ref[...] + bconv_ref[...][None, :], 0.0)
    ps_ref[...] = jnp.sum(z * w2_ref[...][None, :], axis=1) + bh_ref[...]


def _tc2(parts, bconv, w2, bh):
    return pl.pallas_call(
        _tc2_body,
        out_shape=jax.ShapeDtypeStruct((N_SP,), jnp.float32),
    )(parts, bconv, w2, bh)


def _mesh():
    return plsc.VectorSubcoreMesh(
        core_axis_name="c", subcore_axis_name="s",
        num_cores=NC, num_subcores=NS)


_SC_PARAMS = pltpu.CompilerParams(needs_layout_passes=False)


def _sca_body(a_s_hbm, a_d_hbm, src_hbm, dst_hbm, w_hbm, dn_hbm,
              as_v, ad_v, src_v, dst_v, w_v, dn_v):
    wid = lax.axis_index("s") * NC + lax.axis_index("c")
    pltpu.sync_copy(a_s_hbm, as_v)
    pltpu.sync_copy(a_d_hbm, ad_v)
    pltpu.sync_copy(src_hbm.at[wid], src_v)
    pltpu.sync_copy(dst_hbm.at[wid], dst_v)

    zeros = jnp.zeros((16,), jnp.float32)

    def zbody(i, _):
        dn_v[pl.ds(i * 16, 16)] = zeros
        return 0
    lax.fori_loop(0, N_S // 16, zbody, 0)

    def ebody(g, _):
        sl = pl.ds(g * 16, 16)
        sv = src_v[sl]
        dv = dst_v[sl]
        e = plsc.load_gather(as_v, [sv]) + plsc.load_gather(ad_v, [dv])
        e = jnp.where(e >= 0.0, e, 0.2 * e)
        w = jnp.exp(e)
        w_v[sl] = w
        plsc.addupdate_scatter(dn_v, [dv], w)
        return 0
    lax.fori_loop(0, EPT // 16, ebody, 0)

    pltpu.sync_copy(w_v, w_hbm.at[wid])
    pltpu.sync_copy(dn_v, dn_hbm.at[wid])


def _sca(a_s, a_d, src, dst):
    return pl.kernel(
        _sca_body,
        out_type=[
            jax.ShapeDtypeStruct((NW, EPT), jnp.float32),
            jax.ShapeDtypeStruct((NW, N_S), jnp.float32),
        ],
        mesh=_mesh(),
        compiler_params=_SC_PARAMS,
        scratch_types=[
            pltpu.VMEM((N_A,), jnp.float32),
            pltpu.VMEM((N_S,), jnp.float32),
            pltpu.VMEM((EPT,), jnp.int32),
            pltpu.VMEM((EPT,), jnp.int32),
            pltpu.VMEM((EPT,), jnp.float32),
            pltpu.VMEM((N_S,), jnp.float32),
        ],
    )(a_s, a_d, src, dst)


def _scb_body(xs_hbm, r_hbm, src_hbm, dst_hbm, w_hbm, parts_hbm,
              r_v, src_s, dst_s, coef_s, rows_v, zbuf_v, acc, sem):
    sid = lax.axis_index("s")
    pltpu.sync_copy(r_hbm, r_v)

    zeros = jnp.zeros((16,), jnp.float32)

    def zbody(i, _):
        for h in range(H // 16):
            zbuf_v[i, pl.ds(h * 16, 16)] = zeros
        return 0
    lax.fori_loop(0, ZR, zbody, 0)
    base = sid * RPT
    for j in range(RPT // ZR):
        pltpu.sync_copy(zbuf_v, acc.at[pl.ds(base + j * ZR, ZR)])
    plsc.subcore_barrier()

    def sbody(s, _):
        pltpu.sync_copy(src_hbm.at[sid, s], src_s)
        pltpu.sync_copy(dst_hbm.at[sid, s], dst_s)
        pltpu.sync_copy(w_hbm.at[sid, s], coef_s)

        def cbody(c, _):
            for k in range(K // 16):
                sl = pl.ds(k * 16, 16)
                dv = dst_s[c, sl]
                coef_s[c, sl] = coef_s[c, sl] * plsc.load_gather(r_v, [dv])
            return 0
        lax.fori_loop(0, ICH, cbody, 0)

        def ibody(c, _):
            pltpu.async_copy(xs_hbm.at[src_s.at[c]], rows_v, sem).wait()

            c16 = jnp.full((16,), c, jnp.int32)

            def rbody(i, _):
                i16 = jnp.full((16,), i, jnp.int32)
                cv = plsc.load_gather(coef_s, [c16, i16])
                for h in range(H // 16):
                    sl = pl.ds(h * 16, 16)
                    rows_v[i, sl] = rows_v[i, sl] * cv
                return 0
            lax.fori_loop(0, K, rbody, 0)
            pltpu.sync_copy(rows_v, acc.at[dst_s.at[c]], add=True)
            return 0
        lax.fori_loop(0, ICH, ibody, 0)
        return 0
    lax.fori_loop(0, SCH, sbody, 0)

    plsc.subcore_barrier()
    pltpu.sync_copy(acc.at[pl.ds(base, RPT)], parts_hbm.at[pl.ds(base, RPT)])


def _scb(xs, r, src, dst, w):
    return pl.kernel(
        _scb_body,
        out_type=jax.ShapeDtypeStruct((N_SP, H), jnp.float32),
        mesh=plsc.VectorSubcoreMesh(
            core_axis_name="c", subcore_axis_name="s",
            num_cores=1, num_subcores=NS),
        compiler_params=_SC_PARAMS,
        scratch_types=[
            pltpu.VMEM((N_S,), jnp.float32),
            pltpu.VMEM((ICH, K), jnp.int32),
            pltpu.VMEM((ICH, K), jnp.int32),
            pltpu.VMEM((ICH, K), jnp.float32),
            pltpu.VMEM((K, H), jnp.float32),
            pltpu.VMEM((ZR, H), jnp.float32),
            pltpu.VMEM_SHARED((N_SP, H), jnp.float32),
            pltpu.SemaphoreType.DMA,
        ],
    )(xs, r, src, dst, w)


def _scc_body(pa_hbm, ps_hbm, row_hbm, col_hbm, out_hbm,
              pa_v, ps_v, row_v, col_v, o_v):
    wid = lax.axis_index("s") * NC + lax.axis_index("c")
    pltpu.sync_copy(pa_hbm, pa_v)
    pltpu.sync_copy(ps_hbm, ps_v)
    pltpu.sync_copy(row_hbm.at[wid], row_v)
    pltpu.sync_copy(col_hbm.at[wid], col_v)

    def gbody(g, _):
        sl = pl.ds(g * 16, 16)
        rv = row_v[sl]
        cv = col_v[sl]
        logit = plsc.load_gather(pa_v, [rv]) + plsc.load_gather(ps_v, [cv])
        o_v[sl] = 1.0 / (1.0 + jnp.exp(-logit))
        return 0
    lax.fori_loop(0, DPT // 16, gbody, 0)

    pltpu.sync_copy(o_v, out_hbm.at[wid])


def _scc(pa, ps, row, col):
    return pl.kernel(
        _scc_body,
        out_type=jax.ShapeDtypeStruct((NW, DPT), jnp.float32),
        mesh=_mesh(),
        compiler_params=_SC_PARAMS,
        scratch_types=[
            pltpu.VMEM((N_A,), jnp.float32),
            pltpu.VMEM((N_SP,), jnp.float32),
            pltpu.VMEM((DPT,), jnp.int32),
            pltpu.VMEM((DPT,), jnp.int32),
            pltpu.VMEM((DPT,), jnp.float32),
        ],
    )(pa, ps, row, col)


def kernel(x_artwork, x_style, edge_index, edge_label_index,
           W_src, W_dst, att_src, att_dst, b_conv, W_head, b_head):
    src = edge_index[0].astype(jnp.int32)
    dst = edge_index[1].astype(jnp.int32)
    row = edge_label_index[0].astype(jnp.int32)
    col = edge_label_index[1].astype(jnp.int32)

    w1 = W_head[:H, 0]
    w2 = W_head[H:, 0]
    bh = jnp.broadcast_to(b_head, (N_SP,))

    xs, a_s, a_d, p_a = _tc1(x_artwork, x_style, W_src, W_dst,
                             att_src, att_dst, w1)

    src2 = src.reshape(NW, EPT)
    dst2 = dst.reshape(NW, EPT)
    w_e, dn = _sca(a_s, a_d, src2, dst2)

    r = _tcr(dn)

    parts = _scb(xs, r.reshape(N_S),
                 src.reshape(NS, SCH, ICH, K), dst.reshape(NS, SCH, ICH, K),
                 w_e.reshape(NS, SCH, ICH, K))

    ps = _tc2(parts, b_conv, w2, bh)

    pad = jnp.zeros((ELP - EL,), jnp.int32)
    rowp = jnp.concatenate([row, pad]).reshape(NW, DPT)
    colp = jnp.concatenate([col, pad]).reshape(NW, DPT)
    out = _scc(p_a, ps, rowp, colp)

    return out.reshape(ELP)[:EL].reshape(EL, 1)


# speedup vs baseline: 22.1764x; 1.4134x over previous
import jax
import jax.numpy as jnp
from jax import lax
from jax.experimental import pallas as pl
from jax.experimental.pallas import tpu as pltpu
from jax.experimental.pallas import tpu_sc as plsc

N_A = 10000
N_S = 10000
E = 320000
EL = 100000
D = 128
H = 128

NC = 2
NS = 16
NW = NC * NS

EPT = E // NW
EPC = E // NS
K = 80
NCH = EPT // K
NCH2 = EPC // K
SCH = 25
ICH = NCH2 // SCH
SC_E = EPC // SCH
HH = H // 2

ELP = 100352
DPT = ELP // NW
N_SP = 10240
RPT = N_SP // NS
ZR = 32

_HI = lax.Precision.HIGHEST


def _tc1_body(xa_ref, xst_ref, wsrc_ref, wdst_ref, asrc_ref, adst_ref, w1_ref,
              xs_ref, a_s_ref, a_d_ref, p_a_ref):
    xa = xa_ref[...]
    xs = jnp.dot(xa, wsrc_ref[...], precision=_HI)
    xs_ref[...] = xs
    a_s_ref[...] = jnp.sum(xs * asrc_ref[...][None, :], axis=1)
    v = jnp.sum(wdst_ref[...] * adst_ref[...][None, :], axis=1)
    a_d_ref[...] = jnp.sum(xst_ref[...] * v[None, :], axis=1)
    p_a_ref[...] = jnp.sum(xa * w1_ref[...][None, :], axis=1)


def _tc1(xa, xst, wsrc, wdst, asrc, adst, w1):
    return pl.pallas_call(
        _tc1_body,
        out_shape=[
            jax.ShapeDtypeStruct((N_A, H), jnp.float32),
            jax.ShapeDtypeStruct((N_A,), jnp.float32),
            jax.ShapeDtypeStruct((N_S,), jnp.float32),
            jax.ShapeDtypeStruct((N_A,), jnp.float32),
        ],
    )(xa, xst, wsrc, wdst, asrc, adst, w1)


def _tcr_body(dn_ref, r_ref):
    s = jnp.sum(dn_ref[...], axis=0, keepdims=True)
    r_ref[...] = 1.0 / (s + 1e-16)


def _tcr(dn):
    return pl.pallas_call(
        _tcr_body,
        out_shape=jax.ShapeDtypeStruct((1, N_S), jnp.float32),
    )(dn)


def _tc2_body(parts_ref, bconv_ref, w2_ref, bh_ref, ps_ref):
    z = jnp.maximum(parts_ref[...] + bconv_ref[...][None, :], 0.0)
    ps_ref[...] = jnp.sum(z * w2_ref[...][None, :], axis=1) + bh_ref[...]


def _tc2(parts, bconv, w2, bh):
    return pl.pallas_call(
        _tc2_body,
        out_shape=jax.ShapeDtypeStruct((N_SP,), jnp.float32),
    )(parts, bconv, w2, bh)


def _mesh():
    return plsc.VectorSubcoreMesh(
        core_axis_name="c", subcore_axis_name="s",
        num_cores=NC, num_subcores=NS)


_SC_PARAMS = pltpu.CompilerParams(needs_layout_passes=False)


def _sca_body(a_s_hbm, a_d_hbm, src_hbm, dst_hbm, w_hbm, dn_hbm,
              as_v, ad_v, src_v, dst_v, w_v, dn_v):
    wid = lax.axis_index("s") * NC + lax.axis_index("c")
    pltpu.sync_copy(a_s_hbm, as_v)
    pltpu.sync_copy(a_d_hbm, ad_v)
    pltpu.sync_copy(src_hbm.at[wid], src_v)
    pltpu.sync_copy(dst_hbm.at[wid], dst_v)

    zeros = jnp.zeros((16,), jnp.float32)

    def zbody(i, _):
        dn_v[pl.ds(i * 16, 16)] = zeros
        return 0
    lax.fori_loop(0, N_S // 16, zbody, 0)

    def ebody(g, _):
        sl = pl.ds(g * 16, 16)
        sv = src_v[sl]
        dv = dst_v[sl]
        e = plsc.load_gather(as_v, [sv]) + plsc.load_gather(ad_v, [dv])
        e = jnp.where(e >= 0.0, e, 0.2 * e)
        w = jnp.exp(e)
        w_v[sl] = w
        plsc.addupdate_scatter(dn_v, [dv], w)
        return 0
    lax.fori_loop(0, EPT // 16, ebody, 0)

    pltpu.sync_copy(w_v, w_hbm.at[wid])
    pltpu.sync_copy(dn_v, dn_hbm.at[wid])


def _sca(a_s, a_d, src, dst):
    return pl.kernel(
        _sca_body,
        out_type=[
            jax.ShapeDtypeStruct((NW, EPT), jnp.float32),
            jax.ShapeDtypeStruct((NW, N_S), jnp.float32),
        ],
        mesh=_mesh(),
        compiler_params=_SC_PARAMS,
        scratch_types=[
            pltpu.VMEM((N_A,), jnp.float32),
            pltpu.VMEM((N_S,), jnp.float32),
            pltpu.VMEM((EPT,), jnp.int32),
            pltpu.VMEM((EPT,), jnp.int32),
            pltpu.VMEM((EPT,), jnp.float32),
            pltpu.VMEM((N_S,), jnp.float32),
        ],
    )(a_s, a_d, src, dst)


def _scb_body(xs_hbm, r_hbm, src_hbm, dst_hbm, w_hbm, parts_hbm,
              r_v, src_s, dst_s, coef_s, rows_a, rows_b,
              acc, gsa, gsb, ssa, ssb):
    sid = lax.axis_index("s")
    pltpu.sync_copy(r_hbm, r_v)

    zeros = jnp.zeros((16,), jnp.float32)

    def zbody(i, _):
        for h in range(H // 16):
            rows_a[i, pl.ds(h * 16, 16)] = zeros
        return 0
    lax.fori_loop(0, K, zbody, 0)
    base = sid * RPT
    for j in range(RPT // K):
        pltpu.sync_copy(rows_a, acc.at[pl.ds(base + j * K, K)])
    plsc.subcore_barrier()

    bufs = (rows_a, rows_b)
    gsems = (gsa, gsb)
    ssems = (ssa, ssb)

    def _scale(buf, c):
        c16 = jnp.full((16,), c, jnp.int32)

        def rbody(i, _):
            i16 = jnp.full((16,), i, jnp.int32)
            cv = plsc.load_gather(coef_s, [c16, i16])
            for h in range(H // 16):
                sl = pl.ds(h * 16, 16)
                buf[i, sl] = buf[i, sl] * cv
            return 0
        lax.fori_loop(0, K, rbody, 0)

    def sbody(s, _):
        pltpu.sync_copy(src_hbm.at[sid, s], src_s)
        pltpu.sync_copy(dst_hbm.at[sid, s], dst_s)
        pltpu.sync_copy(w_hbm.at[sid, s], coef_s)

        def cbody(c, _):
            for k in range(K // 16):
                sl = pl.ds(k * 16, 16)
                dv = dst_s[c, sl]
                coef_s[c, sl] = coef_s[c, sl] * plsc.load_gather(r_v, [dv])
            return 0
        lax.fori_loop(0, ICH, cbody, 0)

        pltpu.async_copy(xs_hbm.at[src_s.at[0]], rows_a, gsa)

        def ibody(i, _):
            for b in range(2):
                cc = 2 * i + b
                buf, oth = bufs[b], bufs[1 - b]
                pltpu.make_async_copy(
                    xs_hbm.at[src_s.at[cc]], buf, gsems[b]).wait()
                if b == 1:
                    pltpu.make_async_copy(
                        oth, acc.at[dst_s.at[cc - 1]], ssems[0]).wait()
                else:
                    @pl.when(i >= 1)
                    def _():
                        pltpu.make_async_copy(
                            oth, acc.at[dst_s.at[cc - 1]], ssems[1]).wait()
                if b == 0:
                    pltpu.async_copy(
                        xs_hbm.at[src_s.at[cc + 1]], oth, gsems[1])
                else:
                    @pl.when(i < ICH // 2 - 1)
                    def _():
                        pltpu.async_copy(
                            xs_hbm.at[src_s.at[cc + 1]], oth, gsems[0])
                _scale(buf, cc)
                pltpu.async_copy(buf, acc.at[dst_s.at[cc]], ssems[b],
                                 add=True)
            return 0
        lax.fori_loop(0, ICH // 2, ibody, 0)
        pltpu.make_async_copy(
            rows_b, acc.at[dst_s.at[ICH - 1]], ssb).wait()
        return 0
    lax.fori_loop(0, SCH, sbody, 0)

    plsc.subcore_barrier()
    pltpu.sync_copy(acc.at[pl.ds(base, RPT)], parts_hbm.at[pl.ds(base, RPT)])


def _scb(xs, r, src, dst, w):
    return pl.kernel(
        _scb_body,
        out_type=jax.ShapeDtypeStruct((N_SP, H), jnp.float32),
        mesh=plsc.VectorSubcoreMesh(
            core_axis_name="c", subcore_axis_name="s",
            num_cores=1, num_subcores=NS),
        compiler_params=_SC_PARAMS,
        scratch_types=[
            pltpu.VMEM((N_S,), jnp.float32),
            pltpu.VMEM((ICH, K), jnp.int32),
            pltpu.VMEM((ICH, K), jnp.int32),
            pltpu.VMEM((ICH, K), jnp.float32),
            pltpu.VMEM((K, H), jnp.float32),
            pltpu.VMEM((K, H), jnp.float32),
            pltpu.VMEM_SHARED((N_SP, H), jnp.float32),
            pltpu.SemaphoreType.DMA,
            pltpu.SemaphoreType.DMA,
            pltpu.SemaphoreType.DMA,
            pltpu.SemaphoreType.DMA,
        ],
    )(xs, r, src, dst, w)


def _scc_body(pa_hbm, ps_hbm, row_hbm, col_hbm, out_hbm,
              pa_v, ps_v, row_v, col_v, o_v):
    wid = lax.axis_index("s") * NC + lax.axis_index("c")
    pltpu.sync_copy(pa_hbm, pa_v)
    pltpu.sync_copy(ps_hbm, ps_v)
    pltpu.sync_copy(row_hbm.at[wid], row_v)
    pltpu.sync_copy(col_hbm.at[wid], col_v)

    def gbody(g, _):
        sl = pl.ds(g * 16, 16)
        rv = row_v[sl]
        cv = col_v[sl]
        logit = plsc.load_gather(pa_v, [rv]) + plsc.load_gather(ps_v, [cv])
        o_v[sl] = 1.0 / (1.0 + jnp.exp(-logit))
        return 0
    lax.fori_loop(0, DPT // 16, gbody, 0)

    pltpu.sync_copy(o_v, out_hbm.at[wid])


def _scc(pa, ps, row, col):
    return pl.kernel(
        _scc_body,
        out_type=jax.ShapeDtypeStruct((NW, DPT), jnp.float32),
        mesh=_mesh(),
        compiler_params=_SC_PARAMS,
        scratch_types=[
            pltpu.VMEM((N_A,), jnp.float32),
            pltpu.VMEM((N_SP,), jnp.float32),
            pltpu.VMEM((DPT,), jnp.int32),
            pltpu.VMEM((DPT,), jnp.int32),
            pltpu.VMEM((DPT,), jnp.float32),
        ],
    )(pa, ps, row, col)


def kernel(x_artwork, x_style, edge_index, edge_label_index,
           W_src, W_dst, att_src, att_dst, b_conv, W_head, b_head):
    src = edge_index[0].astype(jnp.int32)
    dst = edge_index[1].astype(jnp.int32)
    row = edge_label_index[0].astype(jnp.int32)
    col = edge_label_index[1].astype(jnp.int32)

    w1 = W_head[:H, 0]
    w2 = W_head[H:, 0]
    bh = jnp.broadcast_to(b_head, (N_SP,))

    xs, a_s, a_d, p_a = _tc1(x_artwork, x_style, W_src, W_dst,
                             att_src, att_dst, w1)

    src2 = src.reshape(NW, EPT)
    dst2 = dst.reshape(NW, EPT)
    w_e, dn = _sca(a_s, a_d, src2, dst2)

    r = _tcr(dn)

    parts = _scb(xs, r.reshape(N_S),
                 src.reshape(NS, SCH, ICH, K), dst.reshape(NS, SCH, ICH, K),
                 w_e.reshape(NS, SCH, ICH, K))

    ps = _tc2(parts, b_conv, w2, bh)

    pad = jnp.zeros((ELP - EL,), jnp.int32)
    rowp = jnp.concatenate([row, pad]).reshape(NW, DPT)
    colp = jnp.concatenate([col, pad]).reshape(NW, DPT)
    out = _scc(p_a, ps, rowp, colp)

    return out.reshape(ELP)[:EL].reshape(EL, 1)


# speedup vs baseline: 24.2135x; 1.0919x over previous
import jax
import jax.numpy as jnp
from jax import lax
from jax.experimental import pallas as pl
from jax.experimental.pallas import tpu as pltpu
from jax.experimental.pallas import tpu_sc as plsc

N_A = 10000
N_S = 10000
E = 320000
EL = 100000
D = 128
H = 128

NC = 2
NS = 16
NW = NC * NS

EPT = E // NW
EPC = E // NS
K = 80
NCH = EPT // K
NCH2 = EPC // K
SCH = 25
ICH = NCH2 // SCH
SC_E = EPC // SCH
HH = H // 2

ELP = 100352
DPT = ELP // NW
N_SP = 10240
RPT = N_SP // NS
ZR = 32

_HI = lax.Precision.HIGHEST


def _tc1_body(xa_ref, xst_ref, wsrc_ref, wdst_ref, asrc_ref, adst_ref, w1_ref,
              xs_ref, a_s_ref, a_d_ref, p_a_ref):
    xa = xa_ref[...]
    xs = jnp.dot(xa, wsrc_ref[...], precision=_HI)
    xs_ref[...] = xs
    a_s_ref[...] = jnp.sum(xs * asrc_ref[...][None, :], axis=1)
    v = jnp.sum(wdst_ref[...] * adst_ref[...][None, :], axis=1)
    a_d_ref[...] = jnp.sum(xst_ref[...] * v[None, :], axis=1)
    p_a_ref[...] = jnp.sum(xa * w1_ref[...][None, :], axis=1)


def _tc1(xa, xst, wsrc, wdst, asrc, adst, w1):
    return pl.pallas_call(
        _tc1_body,
        out_shape=[
            jax.ShapeDtypeStruct((N_A, H), jnp.float32),
            jax.ShapeDtypeStruct((N_A,), jnp.float32),
            jax.ShapeDtypeStruct((N_S,), jnp.float32),
            jax.ShapeDtypeStruct((N_A,), jnp.float32),
        ],
    )(xa, xst, wsrc, wdst, asrc, adst, w1)


def _tcr_body(dn_ref, r_ref):
    s = jnp.sum(dn_ref[...], axis=0, keepdims=True)
    r_ref[...] = 1.0 / (s + 1e-16)


def _tcr(dn):
    return pl.pallas_call(
        _tcr_body,
        out_shape=jax.ShapeDtypeStruct((1, N_S), jnp.float32),
    )(dn)


def _tc2_body(parts_ref, bconv_ref, w2_ref, bh_ref, ps_ref):
    z = jnp.maximum(parts_ref[...] + bconv_ref[...][None, :], 0.0)
    ps_ref[...] = jnp.sum(z * w2_ref[...][None, :], axis=1) + bh_ref[...]


def _tc2(parts, bconv, w2, bh):
    return pl.pallas_call(
        _tc2_body,
        out_shape=jax.ShapeDtypeStruct((N_SP,), jnp.float32),
    )(parts, bconv, w2, bh)


def _mesh():
    return plsc.VectorSubcoreMesh(
        core_axis_name="c", subcore_axis_name="s",
        num_cores=NC, num_subcores=NS)


_SC_PARAMS = pltpu.CompilerParams(needs_layout_passes=False)


def _sca_body(a_s_hbm, a_d_hbm, src_hbm, dst_hbm, w_hbm, dn_hbm,
              as_v, ad_v, src_v, dst_v, w_v, dn_v):
    wid = lax.axis_index("s") * NC + lax.axis_index("c")
    pltpu.sync_copy(a_s_hbm, as_v)
    pltpu.sync_copy(a_d_hbm, ad_v)
    pltpu.sync_copy(src_hbm.at[wid], src_v)
    pltpu.sync_copy(dst_hbm.at[wid], dst_v)

    zeros = jnp.zeros((16,), jnp.float32)

    def zbody(i, _):
        dn_v[pl.ds(i * 16, 16)] = zeros
        return 0
    lax.fori_loop(0, N_S // 16, zbody, 0)

    def ebody(g, _):
        sl = pl.ds(g * 16, 16)
        sv = src_v[sl]
        dv = dst_v[sl]
        e = plsc.load_gather(as_v, [sv]) + plsc.load_gather(ad_v, [dv])
        e = jnp.where(e >= 0.0, e, 0.2 * e)
        w = jnp.exp(e)
        w_v[sl] = w
        plsc.addupdate_scatter(dn_v, [dv], w)
        return 0
    lax.fori_loop(0, EPT // 16, ebody, 0)

    pltpu.sync_copy(w_v, w_hbm.at[wid])
    pltpu.sync_copy(dn_v, dn_hbm.at[wid])


def _sca(a_s, a_d, src, dst):
    return pl.kernel(
        _sca_body,
        out_type=[
            jax.ShapeDtypeStruct((NW, EPT), jnp.float32),
            jax.ShapeDtypeStruct((NW, N_S), jnp.float32),
        ],
        mesh=_mesh(),
        compiler_params=_SC_PARAMS,
        scratch_types=[
            pltpu.VMEM((N_A,), jnp.float32),
            pltpu.VMEM((N_S,), jnp.float32),
            pltpu.VMEM((EPT,), jnp.int32),
            pltpu.VMEM((EPT,), jnp.int32),
            pltpu.VMEM((EPT,), jnp.float32),
            pltpu.VMEM((N_S,), jnp.float32),
        ],
    )(a_s, a_d, src, dst)


def _scb_body(xs_hbm, r_hbm, src_hbm, dst_hbm, w_hbm, parts_hbm,
              r_v, src_a, src_b, dst_a, dst_b, w_a, w_b, coef_s,
              rows_a, rows_b, acc, gsa, gsb, ssa, ssb, stsem):
    sid = lax.axis_index("s")
    pltpu.sync_copy(r_hbm, r_v)

    zeros = jnp.zeros((16,), jnp.float32)

    def zbody(i, _):
        for h in range(H // 16):
            rows_a[i, pl.ds(h * 16, 16)] = zeros
        return 0
    lax.fori_loop(0, K, zbody, 0)
    base = sid * RPT
    for j in range(RPT // K):
        pltpu.sync_copy(rows_a, acc.at[pl.ds(base + j * K, K)])
    plsc.subcore_barrier()

    bufs = (rows_a, rows_b)
    gsems = (gsa, gsb)
    ssems = (ssa, ssb)

    def _stage(s, stg):
        src_s, dst_s, w_s = stg
        pltpu.async_copy(src_hbm.at[sid, s], src_s, stsem)
        pltpu.async_copy(dst_hbm.at[sid, s], dst_s, stsem)
        pltpu.async_copy(w_hbm.at[sid, s], w_s, stsem)

    def _stage_wait(s, stg):
        src_s, dst_s, w_s = stg
        pltpu.make_async_copy(src_hbm.at[sid, s], src_s, stsem).wait()
        pltpu.make_async_copy(dst_hbm.at[sid, s], dst_s, stsem).wait()
        pltpu.make_async_copy(w_hbm.at[sid, s], w_s, stsem).wait()

    _stage(0, (src_a, dst_a, w_a))

    def _process(s, stg, stg_next):
        src_s, dst_s, w_s = stg
        _stage_wait(s, stg)

        @pl.when(s + 1 < SCH)
        def _():
            _stage(s + 1, stg_next)

        def cbody(c, _):
            for k in range(K // 16):
                sl = pl.ds(k * 16, 16)
                dv = dst_s[c, sl]
                coef_s[c, sl] = w_s[c, sl] * plsc.load_gather(r_v, [dv])
            return 0
        lax.fori_loop(0, ICH, cbody, 0)

        pltpu.async_copy(xs_hbm.at[src_s.at[0]], rows_a, gsa)

        def ibody(i, _):
            for b in range(2):
                cc = 2 * i + b
                buf, oth = bufs[b], bufs[1 - b]
                pltpu.make_async_copy(
                    xs_hbm.at[src_s.at[cc]], buf, gsems[b]).wait()
                if b == 1:
                    pltpu.make_async_copy(
                        oth, acc.at[dst_s.at[cc - 1]], ssems[0]).wait()
                else:
                    @pl.when(i >= 1)
                    def _():
                        pltpu.make_async_copy(
                            oth, acc.at[dst_s.at[cc - 1]], ssems[1]).wait()
                if b == 0:
                    pltpu.async_copy(
                        xs_hbm.at[src_s.at[cc + 1]], oth, gsems[1])
                else:
                    @pl.when(i < ICH // 2 - 1)
                    def _():
                        pltpu.async_copy(
                            xs_hbm.at[src_s.at[cc + 1]], oth, gsems[0])

                c16 = jnp.full((16,), cc, jnp.int32)

                def rbody(r_i, _):
                    i16 = jnp.full((16,), r_i, jnp.int32)
                    cv = plsc.load_gather(coef_s, [c16, i16])
                    for h in range(H // 16):
                        sl = pl.ds(h * 16, 16)
                        buf[r_i, sl] = buf[r_i, sl] * cv
                    return 0
                lax.fori_loop(0, K, rbody, 0)

                pltpu.async_copy(buf, acc.at[dst_s.at[cc]],
                                 ssems[b], add=True)
            return 0
        lax.fori_loop(0, ICH // 2, ibody, 0)
        pltpu.make_async_copy(
            rows_b, acc.at[dst_s.at[ICH - 1]], ssb).wait()

    def sbody(s, _):
        p = s % 2

        @pl.when(p == 0)
        def _():
            _process(s, (src_a, dst_a, w_a), (src_b, dst_b, w_b))

        @pl.when(p == 1)
        def _():
            _process(s, (src_b, dst_b, w_b), (src_a, dst_a, w_a))
        return 0
    lax.fori_loop(0, SCH, sbody, 0)

    plsc.subcore_barrier()
    pltpu.sync_copy(acc.at[pl.ds(base, RPT)], parts_hbm.at[pl.ds(base, RPT)])


def _scb(xs, r, src, dst, w):
    return pl.kernel(
        _scb_body,
        out_type=jax.ShapeDtypeStruct((N_SP, H), jnp.float32),
        mesh=plsc.VectorSubcoreMesh(
            core_axis_name="c", subcore_axis_name="s",
            num_cores=1, num_subcores=NS),
        compiler_params=_SC_PARAMS,
        scratch_types=[
            pltpu.VMEM((N_S,), jnp.float32),
            pltpu.VMEM((ICH, K), jnp.int32),
            pltpu.VMEM((ICH, K), jnp.int32),
            pltpu.VMEM((ICH, K), jnp.int32),
            pltpu.VMEM((ICH, K), jnp.int32),
            pltpu.VMEM((ICH, K), jnp.float32),
            pltpu.VMEM((ICH, K), jnp.float32),
            pltpu.VMEM((ICH, K), jnp.float32),
            pltpu.VMEM((K, H), jnp.float32),
            pltpu.VMEM((K, H), jnp.float32),
            pltpu.VMEM_SHARED((N_SP, H), jnp.float32),
            pltpu.SemaphoreType.DMA,
            pltpu.SemaphoreType.DMA,
            pltpu.SemaphoreType.DMA,
            pltpu.SemaphoreType.DMA,
            pltpu.SemaphoreType.DMA,
        ],
    )(xs, r, src, dst, w)


def _scc_body(pa_hbm, ps_hbm, row_hbm, col_hbm, out_hbm,
              pa_v, ps_v, row_v, col_v, o_v):
    wid = lax.axis_index("s") * NC + lax.axis_index("c")
    pltpu.sync_copy(pa_hbm, pa_v)
    pltpu.sync_copy(ps_hbm, ps_v)
    pltpu.sync_copy(row_hbm.at[wid], row_v)
    pltpu.sync_copy(col_hbm.at[wid], col_v)

    def gbody(g, _):
        sl = pl.ds(g * 16, 16)
        rv = row_v[sl]
        cv = col_v[sl]
        logit = plsc.load_gather(pa_v, [rv]) + plsc.load_gather(ps_v, [cv])
        o_v[sl] = 1.0 / (1.0 + jnp.exp(-logit))
        return 0
    lax.fori_loop(0, DPT // 16, gbody, 0)

    pltpu.sync_copy(o_v, out_hbm.at[wid])


def _scc(pa, ps, row, col):
    return pl.kernel(
        _scc_body,
        out_type=jax.ShapeDtypeStruct((NW, DPT), jnp.float32),
        mesh=_mesh(),
        compiler_params=_SC_PARAMS,
        scratch_types=[
            pltpu.VMEM((N_A,), jnp.float32),
            pltpu.VMEM((N_SP,), jnp.float32),
            pltpu.VMEM((DPT,), jnp.int32),
            pltpu.VMEM((DPT,), jnp.int32),
            pltpu.VMEM((DPT,), jnp.float32),
        ],
    )(pa, ps, row, col)


def kernel(x_artwork, x_style, edge_index, edge_label_index,
           W_src, W_dst, att_src, att_dst, b_conv, W_head, b_head):
    src = edge_index[0].astype(jnp.int32)
    dst = edge_index[1].astype(jnp.int32)
    row = edge_label_index[0].astype(jnp.int32)
    col = edge_label_index[1].astype(jnp.int32)

    w1 = W_head[:H, 0]
    w2 = W_head[H:, 0]
    bh = jnp.broadcast_to(b_head, (N_SP,))

    xs, a_s, a_d, p_a = _tc1(x_artwork, x_style, W_src, W_dst,
                             att_src, att_dst, w1)

    src2 = src.reshape(NW, EPT)
    dst2 = dst.reshape(NW, EPT)
    w_e, dn = _sca(a_s, a_d, src2, dst2)

    r = _tcr(dn)

    parts = _scb(xs, r.reshape(N_S),
                 src.reshape(NS, SCH, ICH, K), dst.reshape(NS, SCH, ICH, K),
                 w_e.reshape(NS, SCH, ICH, K))

    ps = _tc2(parts, b_conv, w2, bh)

    pad = jnp.zeros((ELP - EL,), jnp.int32)
    rowp = jnp.concatenate([row, pad]).reshape(NW, DPT)
    colp = jnp.concatenate([col, pad]).reshape(NW, DPT)
    out = _scc(p_a, ps, rowp, colp)

    return out.reshape(ELP)[:EL].reshape(EL, 1)


# speedup vs baseline: 24.5161x; 1.0125x over previous
import jax
import jax.numpy as jnp
from jax import lax
from jax.experimental import pallas as pl
from jax.experimental.pallas import tpu as pltpu
from jax.experimental.pallas import tpu_sc as plsc

N_A = 10000
N_S = 10000
E = 320000
EL = 100000
D = 128
H = 128

NC = 2
NS = 16
NW = NC * NS

EPT = E // NW
EPC = E // NS
K = 80
NCH = EPT // K
NCH2 = EPC // K
SCH = 25
ICH = NCH2 // SCH
SC_E = EPC // SCH
HH = H // 2

ELP = 100352
DPT = ELP // NW
ELT = EL - (NW - 1) * DPT
N_SP = 10240
RPT = N_SP // NS
ZR = 32

_HI = lax.Precision.HIGHEST


def _tc1_body(xa_ref, xst_ref, wsrc_ref, wdst_ref, asrc_ref, adst_ref, w1_ref,
              xs_ref, a_s_ref, a_d_ref, p_a_ref):
    xa = xa_ref[...]
    xs = jnp.dot(xa, wsrc_ref[...], precision=_HI)
    xs_ref[...] = xs
    a_s_ref[...] = jnp.sum(xs * asrc_ref[...][None, :], axis=1)
    v = jnp.sum(wdst_ref[...] * adst_ref[...][None, :], axis=1)
    a_d_ref[...] = jnp.sum(xst_ref[...] * v[None, :], axis=1)
    p_a_ref[...] = jnp.sum(xa * w1_ref[...][None, :], axis=1)


def _tc1(xa, xst, wsrc, wdst, asrc, adst, w1):
    return pl.pallas_call(
        _tc1_body,
        out_shape=[
            jax.ShapeDtypeStruct((N_A, H), jnp.float32),
            jax.ShapeDtypeStruct((N_A,), jnp.float32),
            jax.ShapeDtypeStruct((N_S,), jnp.float32),
            jax.ShapeDtypeStruct((N_A,), jnp.float32),
        ],
    )(xa, xst, wsrc, wdst, asrc, adst, w1)


def _tcr_body(dn_ref, r_ref):
    s = jnp.sum(dn_ref[...], axis=0, keepdims=True)
    r_ref[...] = 1.0 / (s + 1e-16)


def _tcr(dn):
    return pl.pallas_call(
        _tcr_body,
        out_shape=jax.ShapeDtypeStruct((1, N_S), jnp.float32),
    )(dn)


def _tc2_body(parts_ref, bconv_ref, w2_ref, bh_ref, ps_ref):
    z = jnp.maximum(parts_ref[...] + bconv_ref[...][None, :], 0.0)
    ps_ref[...] = jnp.sum(z * w2_ref[...][None, :], axis=1) + bh_ref[...]


def _tc2(parts, bconv, w2, bh):
    return pl.pallas_call(
        _tc2_body,
        out_shape=jax.ShapeDtypeStruct((N_SP,), jnp.float32),
    )(parts, bconv, w2, bh)


def _mesh():
    return plsc.VectorSubcoreMesh(
        core_axis_name="c", subcore_axis_name="s",
        num_cores=NC, num_subcores=NS)


_SC_PARAMS = pltpu.CompilerParams(needs_layout_passes=False)


def _sca_body(a_s_hbm, a_d_hbm, src_hbm, dst_hbm, w_hbm, dn_hbm,
              as_v, ad_v, src_v, dst_v, w_v, dn_v, lsem):
    wid = lax.axis_index("s") * NC + lax.axis_index("c")
    pltpu.async_copy(a_s_hbm, as_v, lsem)
    pltpu.async_copy(a_d_hbm, ad_v, lsem)
    pltpu.async_copy(src_hbm.at[wid], src_v, lsem)
    pltpu.async_copy(dst_hbm.at[wid], dst_v, lsem)

    zeros = jnp.zeros((16,), jnp.float32)

    def zbody(i, _):
        dn_v[pl.ds(i * 16, 16)] = zeros
        return 0
    lax.fori_loop(0, N_S // 16, zbody, 0)

    pltpu.make_async_copy(a_s_hbm, as_v, lsem).wait()
    pltpu.make_async_copy(a_d_hbm, ad_v, lsem).wait()
    pltpu.make_async_copy(src_hbm.at[wid], src_v, lsem).wait()
    pltpu.make_async_copy(dst_hbm.at[wid], dst_v, lsem).wait()

    def ebody(g, _):
        sl = pl.ds(g * 16, 16)
        sv = src_v[sl]
        dv = dst_v[sl]
        e = plsc.load_gather(as_v, [sv]) + plsc.load_gather(ad_v, [dv])
        e = jnp.where(e >= 0.0, e, 0.2 * e)
        w = jnp.exp(e)
        w_v[sl] = w
        plsc.addupdate_scatter(dn_v, [dv], w)
        return 0
    lax.fori_loop(0, EPT // 16, ebody, 0)

    pltpu.sync_copy(w_v, w_hbm.at[wid])
    pltpu.sync_copy(dn_v, dn_hbm.at[wid])


def _sca(a_s, a_d, src, dst):
    return pl.kernel(
        _sca_body,
        out_type=[
            jax.ShapeDtypeStruct((NW, EPT), jnp.float32),
            jax.ShapeDtypeStruct((NW, N_S), jnp.float32),
        ],
        mesh=_mesh(),
        compiler_params=_SC_PARAMS,
        scratch_types=[
            pltpu.VMEM((N_A,), jnp.float32),
            pltpu.VMEM((N_S,), jnp.float32),
            pltpu.VMEM((EPT,), jnp.int32),
            pltpu.VMEM((EPT,), jnp.int32),
            pltpu.VMEM((EPT,), jnp.float32),
            pltpu.VMEM((N_S,), jnp.float32),
            pltpu.SemaphoreType.DMA,
        ],
    )(a_s, a_d, src, dst)


def _scb_body(xs_hbm, r_hbm, src_hbm, dst_hbm, w_hbm, parts_hbm,
              r_v, src_a, src_b, dst_a, dst_b, w_a, w_b, coef_s,
              rows_a, rows_b, acc, gsa, gsb, ssa, ssb, stsem):
    sid = lax.axis_index("s")
    pltpu.sync_copy(r_hbm, r_v)

    zeros = jnp.zeros((16,), jnp.float32)

    def zbody(i, _):
        for h in range(H // 16):
            rows_a[i, pl.ds(h * 16, 16)] = zeros
        return 0
    lax.fori_loop(0, K, zbody, 0)
    base = sid * RPT
    for j in range(RPT // K):
        pltpu.sync_copy(rows_a, acc.at[pl.ds(base + j * K, K)])
    plsc.subcore_barrier()

    bufs = (rows_a, rows_b)
    gsems = (gsa, gsb)
    ssems = (ssa, ssb)

    def _stage(s, stg):
        src_s, dst_s, w_s = stg
        pltpu.async_copy(src_hbm.at[sid, s], src_s, stsem)
        pltpu.async_copy(dst_hbm.at[sid, s], dst_s, stsem)
        pltpu.async_copy(w_hbm.at[sid, s], w_s, stsem)

    def _stage_wait(s, stg):
        src_s, dst_s, w_s = stg
        pltpu.make_async_copy(src_hbm.at[sid, s], src_s, stsem).wait()
        pltpu.make_async_copy(dst_hbm.at[sid, s], dst_s, stsem).wait()
        pltpu.make_async_copy(w_hbm.at[sid, s], w_s, stsem).wait()

    _stage(0, (src_a, dst_a, w_a))

    def _process(s, stg, stg_next):
        src_s, dst_s, w_s = stg
        _stage_wait(s, stg)

        @pl.when(s + 1 < SCH)
        def _():
            _stage(s + 1, stg_next)

        def cbody(c, _):
            for k in range(K // 16):
                sl = pl.ds(k * 16, 16)
                dv = dst_s[c, sl]
                coef_s[c, sl] = w_s[c, sl] * plsc.load_gather(r_v, [dv])
            return 0
        lax.fori_loop(0, ICH, cbody, 0)

        pltpu.async_copy(xs_hbm.at[src_s.at[0]], rows_a, gsa)

        def ibody(i, _):
            for b in range(2):
                cc = 2 * i + b
                buf, oth = bufs[b], bufs[1 - b]
                pltpu.make_async_copy(
                    xs_hbm.at[src_s.at[cc]], buf, gsems[b]).wait()
                if b == 1:
                    pltpu.make_async_copy(
                        oth, acc.at[dst_s.at[cc - 1]], ssems[0]).wait()
                else:
                    @pl.when(i >= 1)
                    def _():
                        pltpu.make_async_copy(
                            oth, acc.at[dst_s.at[cc - 1]], ssems[1]).wait()
                if b == 0:
                    pltpu.async_copy(
                        xs_hbm.at[src_s.at[cc + 1]], oth, gsems[1])
                else:
                    @pl.when(i < ICH // 2 - 1)
                    def _():
                        pltpu.async_copy(
                            xs_hbm.at[src_s.at[cc + 1]], oth, gsems[0])

                c16 = jnp.full((16,), cc, jnp.int32)

                def rbody(r_i, _):
                    i16 = jnp.full((16,), r_i, jnp.int32)
                    cv = plsc.load_gather(coef_s, [c16, i16])
                    for h in range(H // 16):
                        sl = pl.ds(h * 16, 16)
                        buf[r_i, sl] = buf[r_i, sl] * cv
                    return 0
                lax.fori_loop(0, K, rbody, 0)

                pltpu.async_copy(buf, acc.at[dst_s.at[cc]],
                                 ssems[b], add=True)
            return 0
        lax.fori_loop(0, ICH // 2, ibody, 0)
        pltpu.make_async_copy(
            rows_b, acc.at[dst_s.at[ICH - 1]], ssb).wait()

    def sbody(s, _):
        p = s % 2

        @pl.when(p == 0)
        def _():
            _process(s, (src_a, dst_a, w_a), (src_b, dst_b, w_b))

        @pl.when(p == 1)
        def _():
            _process(s, (src_b, dst_b, w_b), (src_a, dst_a, w_a))
        return 0
    lax.fori_loop(0, SCH, sbody, 0)

    plsc.subcore_barrier()
    pltpu.sync_copy(acc.at[pl.ds(base, RPT)], parts_hbm.at[pl.ds(base, RPT)])


def _scb(xs, r, src, dst, w):
    return pl.kernel(
        _scb_body,
        out_type=jax.ShapeDtypeStruct((N_SP, H), jnp.float32),
        mesh=plsc.VectorSubcoreMesh(
            core_axis_name="c", subcore_axis_name="s",
            num_cores=1, num_subcores=NS),
        compiler_params=_SC_PARAMS,
        scratch_types=[
            pltpu.VMEM((N_S,), jnp.float32),
            pltpu.VMEM((ICH, K), jnp.int32),
            pltpu.VMEM((ICH, K), jnp.int32),
            pltpu.VMEM((ICH, K), jnp.int32),
            pltpu.VMEM((ICH, K), jnp.int32),
            pltpu.VMEM((ICH, K), jnp.float32),
            pltpu.VMEM((ICH, K), jnp.float32),
            pltpu.VMEM((ICH, K), jnp.float32),
            pltpu.VMEM((K, H), jnp.float32),
            pltpu.VMEM((K, H), jnp.float32),
            pltpu.VMEM_SHARED((N_SP, H), jnp.float32),
            pltpu.SemaphoreType.DMA,
            pltpu.SemaphoreType.DMA,
            pltpu.SemaphoreType.DMA,
            pltpu.SemaphoreType.DMA,
            pltpu.SemaphoreType.DMA,
        ],
    )(xs, r, src, dst, w)


def _scc_body(pa_hbm, ps_hbm, row_hbm, col_hbm, out_hbm,
              pa_v, ps_v, row_v, col_v, o_v, lsem):
    wid = lax.axis_index("s") * NC + lax.axis_index("c")
    start = wid * DPT
    pltpu.async_copy(pa_hbm, pa_v, lsem)
    pltpu.async_copy(ps_hbm, ps_v, lsem)
    pltpu.async_copy(row_hbm.at[pl.ds(start, ELT)], row_v.at[pl.ds(0, ELT)],
                     lsem)
    pltpu.async_copy(col_hbm.at[pl.ds(start, ELT)], col_v.at[pl.ds(0, ELT)],
                     lsem)

    @pl.when(wid < NW - 1)
    def _():
        pltpu.async_copy(row_hbm.at[pl.ds(start + ELT, DPT - ELT)],
                         row_v.at[pl.ds(ELT, DPT - ELT)], lsem)
        pltpu.async_copy(col_hbm.at[pl.ds(start + ELT, DPT - ELT)],
                         col_v.at[pl.ds(ELT, DPT - ELT)], lsem)
        pltpu.make_async_copy(row_hbm.at[pl.ds(start + ELT, DPT - ELT)],
                              row_v.at[pl.ds(ELT, DPT - ELT)], lsem).wait()
        pltpu.make_async_copy(col_hbm.at[pl.ds(start + ELT, DPT - ELT)],
                              col_v.at[pl.ds(ELT, DPT - ELT)], lsem).wait()

    pltpu.make_async_copy(pa_hbm, pa_v, lsem).wait()
    pltpu.make_async_copy(ps_hbm, ps_v, lsem).wait()
    pltpu.make_async_copy(row_hbm.at[pl.ds(start, ELT)],
                          row_v.at[pl.ds(0, ELT)], lsem).wait()
    pltpu.make_async_copy(col_hbm.at[pl.ds(start, ELT)],
                          col_v.at[pl.ds(0, ELT)], lsem).wait()

    zeros16 = jnp.zeros((16,), jnp.int32)

    @pl.when(wid == NW - 1)
    def _():
        def tzero(i, _):
            sl = pl.ds(ELT + i * 16, 16)
            row_v[sl] = zeros16
            col_v[sl] = zeros16
            return 0
        lax.fori_loop(0, (DPT - ELT) // 16, tzero, 0)

    def gbody(g, _):
        sl = pl.ds(g * 16, 16)
        rv = row_v[sl]
        cv = col_v[sl]
        logit = plsc.load_gather(pa_v, [rv]) + plsc.load_gather(ps_v, [cv])
        o_v[sl] = 1.0 / (1.0 + jnp.exp(-logit))
        return 0
    lax.fori_loop(0, DPT // 16, gbody, 0)

    pltpu.sync_copy(o_v, out_hbm.at[wid])


def _scc(pa, ps, row, col):
    return pl.kernel(
        _scc_body,
        out_type=jax.ShapeDtypeStruct((NW, DPT), jnp.float32),
        mesh=_mesh(),
        compiler_params=_SC_PARAMS,
        scratch_types=[
            pltpu.VMEM((N_A,), jnp.float32),
            pltpu.VMEM((N_SP,), jnp.float32),
            pltpu.VMEM((DPT,), jnp.int32),
            pltpu.VMEM((DPT,), jnp.int32),
            pltpu.VMEM((DPT,), jnp.float32),
            pltpu.SemaphoreType.DMA,
        ],
    )(pa, ps, row, col)


def kernel(x_artwork, x_style, edge_index, edge_label_index,
           W_src, W_dst, att_src, att_dst, b_conv, W_head, b_head):
    src = edge_index[0].astype(jnp.int32)
    dst = edge_index[1].astype(jnp.int32)
    row = edge_label_index[0].astype(jnp.int32)
    col = edge_label_index[1].astype(jnp.int32)

    w1 = W_head[:H, 0]
    w2 = W_head[H:, 0]
    bh = jnp.broadcast_to(b_head, (N_SP,))

    xs, a_s, a_d, p_a = _tc1(x_artwork, x_style, W_src, W_dst,
                             att_src, att_dst, w1)

    src2 = src.reshape(NW, EPT)
    dst2 = dst.reshape(NW, EPT)
    w_e, dn = _sca(a_s, a_d, src2, dst2)

    r = _tcr(dn)

    parts = _scb(xs, r.reshape(N_S),
                 src.reshape(NS, SCH, ICH, K), dst.reshape(NS, SCH, ICH, K),
                 w_e.reshape(NS, SCH, ICH, K))

    ps = _tc2(parts, b_conv, w2, bh)

    out = _scc(p_a, ps, row, col)

    return out.reshape(ELP)[:EL].reshape(EL, 1)


# speedup vs baseline: 24.9928x; 1.0194x over previous
import jax
import jax.numpy as jnp
from jax import lax
from jax.experimental import pallas as pl
from jax.experimental.pallas import tpu as pltpu
from jax.experimental.pallas import tpu_sc as plsc

N_A = 10000
N_S = 10000
E = 320000
EL = 100000
D = 128
H = 128

NC = 2
NS = 16
NW = NC * NS

EPT = E // NW
EPC = E // NS
K = 80
NCH = EPT // K
NCH2 = EPC // K
SCH = 25
ICH = NCH2 // SCH
SC_E = EPC // SCH
HH = H // 2

ELP = 100352
DPT = ELP // NW
ELT = EL - (NW - 1) * DPT
N_SP = 10240
RPT = N_SP // NS
ZR = 32

_HI = lax.Precision.HIGHEST


def _tc1_body(xa_ref, xst_ref, wsrc_ref, wdst_ref, asrc_ref, adst_ref, w1_ref,
              xs_ref, a_s_ref, a_d_ref, p_a_ref):
    xa = xa_ref[...]
    xs = jnp.dot(xa, wsrc_ref[...], precision=_HI)
    xs_ref[...] = xs
    a_s_ref[...] = jnp.sum(xs * asrc_ref[...][None, :], axis=1)
    v = jnp.sum(wdst_ref[...] * adst_ref[...][None, :], axis=1)
    a_d_ref[...] = jnp.sum(xst_ref[...] * v[None, :], axis=1)
    p_a_ref[...] = jnp.sum(xa * w1_ref[...][None, :], axis=1)


def _tc1(xa, xst, wsrc, wdst, asrc, adst, w1):
    return pl.pallas_call(
        _tc1_body,
        out_shape=[
            jax.ShapeDtypeStruct((N_A, H), jnp.float32),
            jax.ShapeDtypeStruct((N_A,), jnp.float32),
            jax.ShapeDtypeStruct((N_S,), jnp.float32),
            jax.ShapeDtypeStruct((N_A,), jnp.float32),
        ],
    )(xa, xst, wsrc, wdst, asrc, adst, w1)


def _tc2_body(parts_ref, dn_ref, bconv_ref, w2_ref, bh_ref, ps_ref):
    s = jnp.sum(dn_ref[...], axis=0)
    r = 1.0 / (s + 1e-16)
    z = jnp.maximum(parts_ref[...] * r[:, None] + bconv_ref[...][None, :], 0.0)
    ps_ref[...] = jnp.sum(z * w2_ref[...][None, :], axis=1) + bh_ref[...]


def _tc2(parts, dn, bconv, w2, bh):
    return pl.pallas_call(
        _tc2_body,
        out_shape=jax.ShapeDtypeStruct((N_SP,), jnp.float32),
    )(parts, dn, bconv, w2, bh)


def _mesh():
    return plsc.VectorSubcoreMesh(
        core_axis_name="c", subcore_axis_name="s",
        num_cores=NC, num_subcores=NS)


_SC_PARAMS = pltpu.CompilerParams(needs_layout_passes=False)


def _sca_body(a_s_hbm, a_d_hbm, src_hbm, dst_hbm, w_hbm, dn_hbm,
              as_v, ad_v, src_v, dst_v, w_v, dn_v, lsem):
    wid = lax.axis_index("s") * NC + lax.axis_index("c")
    pltpu.async_copy(a_s_hbm, as_v, lsem)
    pltpu.async_copy(a_d_hbm, ad_v, lsem)
    pltpu.async_copy(src_hbm.at[wid], src_v, lsem)
    pltpu.async_copy(dst_hbm.at[wid], dst_v, lsem)

    zeros = jnp.zeros((16,), jnp.float32)

    def zbody(i, _):
        dn_v[pl.ds(i * 16, 16)] = zeros
        return 0
    lax.fori_loop(0, N_SP // 16, zbody, 0)

    pltpu.make_async_copy(a_s_hbm, as_v, lsem).wait()
    pltpu.make_async_copy(a_d_hbm, ad_v, lsem).wait()
    pltpu.make_async_copy(src_hbm.at[wid], src_v, lsem).wait()
    pltpu.make_async_copy(dst_hbm.at[wid], dst_v, lsem).wait()

    def ebody(g, _):
        sl = pl.ds(g * 16, 16)
        sv = src_v[sl]
        dv = dst_v[sl]
        e = plsc.load_gather(as_v, [sv]) + plsc.load_gather(ad_v, [dv])
        e = jnp.where(e >= 0.0, e, 0.2 * e)
        w = jnp.exp(e)
        w_v[sl] = w
        plsc.addupdate_scatter(dn_v, [dv], w)
        return 0
    lax.fori_loop(0, EPT // 16, ebody, 0)

    pltpu.sync_copy(w_v, w_hbm.at[wid])
    pltpu.sync_copy(dn_v, dn_hbm.at[wid])


def _sca(a_s, a_d, src, dst):
    return pl.kernel(
        _sca_body,
        out_type=[
            jax.ShapeDtypeStruct((NW, EPT), jnp.float32),
            jax.ShapeDtypeStruct((NW, N_SP), jnp.float32),
        ],
        mesh=_mesh(),
        compiler_params=_SC_PARAMS,
        scratch_types=[
            pltpu.VMEM((N_A,), jnp.float32),
            pltpu.VMEM((N_S,), jnp.float32),
            pltpu.VMEM((EPT,), jnp.int32),
            pltpu.VMEM((EPT,), jnp.int32),
            pltpu.VMEM((EPT,), jnp.float32),
            pltpu.VMEM((N_SP,), jnp.float32),
            pltpu.SemaphoreType.DMA,
        ],
    )(a_s, a_d, src, dst)


def _scb_body(xs_hbm, src_hbm, dst_hbm, w_hbm, parts_hbm,
              src_a, src_b, dst_a, dst_b, w_a, w_b,
              rows_a, rows_b, acc, gsa, gsb, ssa, ssb, stsem):
    sid = lax.axis_index("s")

    zeros = jnp.zeros((16,), jnp.float32)

    def zbody(i, _):
        for h in range(H // 16):
            rows_a[i, pl.ds(h * 16, 16)] = zeros
        return 0
    lax.fori_loop(0, K, zbody, 0)
    base = sid * RPT
    for j in range(RPT // K):
        pltpu.sync_copy(rows_a, acc.at[pl.ds(base + j * K, K)])
    plsc.subcore_barrier()

    bufs = (rows_a, rows_b)
    gsems = (gsa, gsb)
    ssems = (ssa, ssb)

    def _stage(s, stg):
        src_s, dst_s, w_s = stg
        pltpu.async_copy(src_hbm.at[sid, s], src_s, stsem)
        pltpu.async_copy(dst_hbm.at[sid, s], dst_s, stsem)
        pltpu.async_copy(w_hbm.at[sid, s], w_s, stsem)

    def _stage_wait(s, stg):
        src_s, dst_s, w_s = stg
        pltpu.make_async_copy(src_hbm.at[sid, s], src_s, stsem).wait()
        pltpu.make_async_copy(dst_hbm.at[sid, s], dst_s, stsem).wait()
        pltpu.make_async_copy(w_hbm.at[sid, s], w_s, stsem).wait()

    _stage(0, (src_a, dst_a, w_a))

    def _process(s, stg, stg_next):
        src_s, dst_s, w_s = stg
        _stage_wait(s, stg)

        @pl.when(s + 1 < SCH)
        def _():
            _stage(s + 1, stg_next)

        pltpu.async_copy(xs_hbm.at[src_s.at[0]], rows_a, gsa)

        def ibody(i, _):
            for b in range(2):
                cc = 2 * i + b
                buf, oth = bufs[b], bufs[1 - b]
                pltpu.make_async_copy(
                    xs_hbm.at[src_s.at[cc]], buf, gsems[b]).wait()
                if b == 1:
                    pltpu.make_async_copy(
                        oth, acc.at[dst_s.at[cc - 1]], ssems[0]).wait()
                else:
                    @pl.when(i >= 1)
                    def _():
                        pltpu.make_async_copy(
                            oth, acc.at[dst_s.at[cc - 1]], ssems[1]).wait()
                if b == 0:
                    pltpu.async_copy(
                        xs_hbm.at[src_s.at[cc + 1]], oth, gsems[1])
                else:
                    @pl.when(i < ICH // 2 - 1)
                    def _():
                        pltpu.async_copy(
                            xs_hbm.at[src_s.at[cc + 1]], oth, gsems[0])

                c16 = jnp.full((16,), cc, jnp.int32)

                def rbody(r_i, _):
                    i16 = jnp.full((16,), r_i, jnp.int32)
                    cv = plsc.load_gather(w_s, [c16, i16])
                    for h in range(H // 16):
                        sl = pl.ds(h * 16, 16)
                        buf[r_i, sl] = buf[r_i, sl] * cv
                    return 0
                lax.fori_loop(0, K, rbody, 0)

                pltpu.async_copy(buf, acc.at[dst_s.at[cc]],
                                 ssems[b], add=True)
            return 0
        lax.fori_loop(0, ICH // 2, ibody, 0)
        pltpu.make_async_copy(
            rows_b, acc.at[dst_s.at[ICH - 1]], ssb).wait()

    def sbody(s, _):
        p = s % 2

        @pl.when(p == 0)
        def _():
            _process(s, (src_a, dst_a, w_a), (src_b, dst_b, w_b))

        @pl.when(p == 1)
        def _():
            _process(s, (src_b, dst_b, w_b), (src_a, dst_a, w_a))
        return 0
    lax.fori_loop(0, SCH, sbody, 0)

    plsc.subcore_barrier()
    pltpu.sync_copy(acc.at[pl.ds(base, RPT)], parts_hbm.at[pl.ds(base, RPT)])


def _scb(xs, src, dst, w):
    return pl.kernel(
        _scb_body,
        out_type=jax.ShapeDtypeStruct((N_SP, H), jnp.float32),
        mesh=plsc.VectorSubcoreMesh(
            core_axis_name="c", subcore_axis_name="s",
            num_cores=1, num_subcores=NS),
        compiler_params=_SC_PARAMS,
        scratch_types=[
            pltpu.VMEM((ICH, K), jnp.int32),
            pltpu.VMEM((ICH, K), jnp.int32),
            pltpu.VMEM((ICH, K), jnp.int32),
            pltpu.VMEM((ICH, K), jnp.int32),
            pltpu.VMEM((ICH, K), jnp.float32),
            pltpu.VMEM((ICH, K), jnp.float32),
            pltpu.VMEM((K, H), jnp.float32),
            pltpu.VMEM((K, H), jnp.float32),
            pltpu.VMEM_SHARED((N_SP, H), jnp.float32),
            pltpu.SemaphoreType.DMA,
            pltpu.SemaphoreType.DMA,
            pltpu.SemaphoreType.DMA,
            pltpu.SemaphoreType.DMA,
            pltpu.SemaphoreType.DMA,
        ],
    )(xs, src, dst, w)


def _scc_body(pa_hbm, ps_hbm, row_hbm, col_hbm, out_hbm,
              pa_v, ps_v, row_v, col_v, o_v, lsem):
    wid = lax.axis_index("s") * NC + lax.axis_index("c")
    start = wid * DPT
    pltpu.async_copy(pa_hbm, pa_v, lsem)
    pltpu.async_copy(ps_hbm, ps_v, lsem)
    pltpu.async_copy(row_hbm.at[pl.ds(start, ELT)], row_v.at[pl.ds(0, ELT)],
                     lsem)
    pltpu.async_copy(col_hbm.at[pl.ds(start, ELT)], col_v.at[pl.ds(0, ELT)],
                     lsem)

    @pl.when(wid < NW - 1)
    def _():
        pltpu.async_copy(row_hbm.at[pl.ds(start + ELT, DPT - ELT)],
                         row_v.at[pl.ds(ELT, DPT - ELT)], lsem)
        pltpu.async_copy(col_hbm.at[pl.ds(start + ELT, DPT - ELT)],
                         col_v.at[pl.ds(ELT, DPT - ELT)], lsem)
        pltpu.make_async_copy(row_hbm.at[pl.ds(start + ELT, DPT - ELT)],
                              row_v.at[pl.ds(ELT, DPT - ELT)], lsem).wait()
        pltpu.make_async_copy(col_hbm.at[pl.ds(start + ELT, DPT - ELT)],
                              col_v.at[pl.ds(ELT, DPT - ELT)], lsem).wait()

    pltpu.make_async_copy(pa_hbm, pa_v, lsem).wait()
    pltpu.make_async_copy(ps_hbm, ps_v, lsem).wait()
    pltpu.make_async_copy(row_hbm.at[pl.ds(start, ELT)],
                          row_v.at[pl.ds(0, ELT)], lsem).wait()
    pltpu.make_async_copy(col_hbm.at[pl.ds(start, ELT)],
                          col_v.at[pl.ds(0, ELT)], lsem).wait()

    zeros16 = jnp.zeros((16,), jnp.int32)

    @pl.when(wid == NW - 1)
    def _():
        def tzero(i, _):
            sl = pl.ds(ELT + i * 16, 16)
            row_v[sl] = zeros16
            col_v[sl] = zeros16
            return 0
        lax.fori_loop(0, (DPT - ELT) // 16, tzero, 0)

    def gbody(g, _):
        sl = pl.ds(g * 16, 16)
        rv = row_v[sl]
        cv = col_v[sl]
        logit = plsc.load_gather(pa_v, [rv]) + plsc.load_gather(ps_v, [cv])
        o_v[sl] = 1.0 / (1.0 + jnp.exp(-logit))
        return 0
    lax.fori_loop(0, DPT // 16, gbody, 0)

    pltpu.sync_copy(o_v, out_hbm.at[wid])


def _scc(pa, ps, row, col):
    return pl.kernel(
        _scc_body,
        out_type=jax.ShapeDtypeStruct((NW, DPT), jnp.float32),
        mesh=_mesh(),
        compiler_params=_SC_PARAMS,
        scratch_types=[
            pltpu.VMEM((N_A,), jnp.float32),
            pltpu.VMEM((N_SP,), jnp.float32),
            pltpu.VMEM((DPT,), jnp.int32),
            pltpu.VMEM((DPT,), jnp.int32),
            pltpu.VMEM((DPT,), jnp.float32),
            pltpu.SemaphoreType.DMA,
        ],
    )(pa, ps, row, col)


def kernel(x_artwork, x_style, edge_index, edge_label_index,
           W_src, W_dst, att_src, att_dst, b_conv, W_head, b_head):
    src = edge_index[0].astype(jnp.int32)
    dst = edge_index[1].astype(jnp.int32)
    row = edge_label_index[0].astype(jnp.int32)
    col = edge_label_index[1].astype(jnp.int32)

    w1 = W_head[:H, 0]
    w2 = W_head[H:, 0]
    bh = jnp.broadcast_to(b_head, (N_SP,))

    xs, a_s, a_d, p_a = _tc1(x_artwork, x_style, W_src, W_dst,
                             att_src, att_dst, w1)

    src2 = src.reshape(NW, EPT)
    dst2 = dst.reshape(NW, EPT)
    w_e, dn = _sca(a_s, a_d, src2, dst2)

    parts = _scb(xs,
                 src.reshape(NS, SCH, ICH, K), dst.reshape(NS, SCH, ICH, K),
                 w_e.reshape(NS, SCH, ICH, K))

    ps = _tc2(parts, dn, b_conv, w2, bh)

    out = _scc(p_a, ps, row, col)

    return out.reshape(ELP)[:EL].reshape(EL, 1)


# speedup vs baseline: 25.8766x; 1.0354x over previous
import jax
import jax.numpy as jnp
from jax import lax
from jax.experimental import pallas as pl
from jax.experimental.pallas import tpu as pltpu
from jax.experimental.pallas import tpu_sc as plsc

N_A = 10000
N_S = 10000
E = 320000
EL = 100000
D = 128
H = 128

NC = 2
NS = 16
NW = NC * NS

EPT = E // NW
EPC = E // NS
K = 80
NCH = EPT // K
NCH2 = EPC // K
SCH = 25
ICH = NCH2 // SCH
SC_E = EPC // SCH
HH = H // 2

ELP = 100352
DPT = ELP // NW
ELT = EL - (NW - 1) * DPT
N_SP = 10240
RPT = N_SP // NS
ZR = 32

_HI = lax.Precision.HIGHEST


def _tc1_body(xa_ref, xst_ref, wsrc_ref, wdst_ref, asrc_ref, adst_ref, w1_ref,
              xs_ref, a_s_ref, a_d_ref, p_a_ref):
    xa = xa_ref[...]
    xs = jnp.dot(xa, wsrc_ref[...], precision=_HI)
    xs_ref[...] = xs
    a_s_ref[...] = jnp.sum(xs * asrc_ref[...][None, :], axis=1)
    v = jnp.sum(wdst_ref[...] * adst_ref[...][None, :], axis=1)
    a_d_ref[...] = jnp.sum(xst_ref[...] * v[None, :], axis=1)
    p_a_ref[...] = jnp.sum(xa * w1_ref[...][None, :], axis=1)


def _tc1(xa, xst, wsrc, wdst, asrc, adst, w1):
    return pl.pallas_call(
        _tc1_body,
        out_shape=[
            jax.ShapeDtypeStruct((N_A, H), jnp.float32),
            jax.ShapeDtypeStruct((N_A,), jnp.float32),
            jax.ShapeDtypeStruct((N_S,), jnp.float32),
            jax.ShapeDtypeStruct((N_A,), jnp.float32),
        ],
    )(xa, xst, wsrc, wdst, asrc, adst, w1)


def _tc2_body(parts_ref, dn_ref, bconv_ref, w2_ref, bh_ref, ps_ref):
    s = jnp.sum(dn_ref[...], axis=0)
    r = 1.0 / (s + 1e-16)
    z = jnp.maximum(parts_ref[...] * r[:, None] + bconv_ref[...][None, :], 0.0)
    ps_ref[...] = jnp.sum(z * w2_ref[...][None, :], axis=1) + bh_ref[...]


def _tc2(parts, dn, bconv, w2, bh):
    return pl.pallas_call(
        _tc2_body,
        out_shape=jax.ShapeDtypeStruct((N_SP,), jnp.float32),
    )(parts, dn, bconv, w2, bh)


def _mesh():
    return plsc.VectorSubcoreMesh(
        core_axis_name="c", subcore_axis_name="s",
        num_cores=NC, num_subcores=NS)


_SC_PARAMS = pltpu.CompilerParams(needs_layout_passes=False)


def _sca_body(a_s_hbm, a_d_hbm, src_hbm, dst_hbm, w_hbm, dn_hbm,
              as_v, ad_v, src_v, dst_v, w_v, dn_v, lsem):
    wid = lax.axis_index("s") * NC + lax.axis_index("c")
    pltpu.async_copy(a_s_hbm, as_v, lsem)
    pltpu.async_copy(a_d_hbm, ad_v, lsem)
    pltpu.async_copy(src_hbm.at[wid], src_v, lsem)
    pltpu.async_copy(dst_hbm.at[wid], dst_v, lsem)

    zeros = jnp.zeros((16,), jnp.float32)

    def zbody(i, _):
        dn_v[pl.ds(i * 16, 16)] = zeros
        return 0
    lax.fori_loop(0, N_SP // 16, zbody, 0)

    pltpu.make_async_copy(a_s_hbm, as_v, lsem).wait()
    pltpu.make_async_copy(a_d_hbm, ad_v, lsem).wait()
    pltpu.make_async_copy(src_hbm.at[wid], src_v, lsem).wait()
    pltpu.make_async_copy(dst_hbm.at[wid], dst_v, lsem).wait()

    def ebody(g, _):
        sl = pl.ds(g * 16, 16)
        sv = src_v[sl]
        dv = dst_v[sl]
        e = plsc.load_gather(as_v, [sv]) + plsc.load_gather(ad_v, [dv])
        e = jnp.where(e >= 0.0, e, 0.2 * e)
        w = jnp.exp(e)
        w_v[sl] = w
        plsc.addupdate_scatter(dn_v, [dv], w)
        return 0
    lax.fori_loop(0, EPT // 16, ebody, 0)

    pltpu.sync_copy(w_v, w_hbm.at[wid])
    pltpu.sync_copy(dn_v, dn_hbm.at[wid])


def _sca(a_s, a_d, src, dst):
    return pl.kernel(
        _sca_body,
        out_type=[
            jax.ShapeDtypeStruct((NW, EPT), jnp.float32),
            jax.ShapeDtypeStruct((NW, N_SP), jnp.float32),
        ],
        mesh=_mesh(),
        compiler_params=_SC_PARAMS,
        scratch_types=[
            pltpu.VMEM((N_A,), jnp.float32),
            pltpu.VMEM((N_S,), jnp.float32),
            pltpu.VMEM((EPT,), jnp.int32),
            pltpu.VMEM((EPT,), jnp.int32),
            pltpu.VMEM((EPT,), jnp.float32),
            pltpu.VMEM((N_SP,), jnp.float32),
            pltpu.SemaphoreType.DMA,
        ],
    )(a_s, a_d, src, dst)


def _scb_body(xs_hbm, src_hbm, dst_hbm, w_hbm, parts_hbm,
              src_a, src_b, dst_a, dst_b, w_a, w_b,
              rows_a, rows_b, acc, gsa, gsb, ssa, ssb, stsem):
    sid = lax.axis_index("s")

    zeros = jnp.zeros((16,), jnp.float32)

    def zbody(i, _):
        for h in range(H // 16):
            rows_a[i, pl.ds(h * 16, 16)] = zeros
        return 0
    lax.fori_loop(0, K, zbody, 0)
    base = sid * RPT
    for j in range(RPT // K):
        pltpu.sync_copy(rows_a, acc.at[pl.ds(base + j * K, K)])
    plsc.subcore_barrier()

    bufs = (rows_a, rows_b)
    gsems = (gsa, gsb)
    ssems = (ssa, ssb)

    def _stage(s, stg):
        src_s, dst_s, w_s = stg
        pltpu.async_copy(src_hbm.at[sid, s], src_s, stsem)
        pltpu.async_copy(dst_hbm.at[sid, s], dst_s, stsem)
        pltpu.async_copy(w_hbm.at[sid, s], w_s, stsem)

    def _stage_wait(s, stg):
        src_s, dst_s, w_s = stg
        pltpu.make_async_copy(src_hbm.at[sid, s], src_s, stsem).wait()
        pltpu.make_async_copy(dst_hbm.at[sid, s], dst_s, stsem).wait()
        pltpu.make_async_copy(w_hbm.at[sid, s], w_s, stsem).wait()

    _stage(0, (src_a, dst_a, w_a))

    def _process(s, stg, stg_next):
        src_s, dst_s, w_s = stg
        _stage_wait(s, stg)

        @pl.when(s + 1 < SCH)
        def _():
            _stage(s + 1, stg_next)

        pltpu.async_copy(xs_hbm.at[src_s.at[0]], rows_a, gsa)

        def ibody(i, _):
            for b in range(2):
                cc = 2 * i + b
                buf, oth = bufs[b], bufs[1 - b]
                pltpu.make_async_copy(
                    xs_hbm.at[src_s.at[cc]], buf, gsems[b]).wait()
                if b == 1:
                    pltpu.make_async_copy(
                        oth, acc.at[dst_s.at[cc - 1]], ssems[0]).wait()
                else:
                    @pl.when(i >= 1)
                    def _():
                        pltpu.make_async_copy(
                            oth, acc.at[dst_s.at[cc - 1]], ssems[1]).wait()
                if b == 0:
                    pltpu.async_copy(
                        xs_hbm.at[src_s.at[cc + 1]], oth, gsems[1])
                else:
                    @pl.when(i < ICH // 2 - 1)
                    def _():
                        pltpu.async_copy(
                            xs_hbm.at[src_s.at[cc + 1]], oth, gsems[0])

                c16 = jnp.full((16,), cc, jnp.int32)

                def rbody(r_g, _):
                    for u in range(2):
                        r_i = r_g * 2 + u
                        i16 = jnp.full((16,), r_i, jnp.int32)
                        cv = plsc.load_gather(w_s, [c16, i16])
                        for h in range(H // 16):
                            sl = pl.ds(h * 16, 16)
                            buf[r_i, sl] = buf[r_i, sl] * cv
                    return 0
                lax.fori_loop(0, K // 2, rbody, 0)

                pltpu.async_copy(buf, acc.at[dst_s.at[cc]],
                                 ssems[b], add=True)
            return 0
        lax.fori_loop(0, ICH // 2, ibody, 0)
        pltpu.make_async_copy(
            rows_b, acc.at[dst_s.at[ICH - 1]], ssb).wait()

    def sbody(s, _):
        p = s % 2

        @pl.when(p == 0)
        def _():
            _process(s, (src_a, dst_a, w_a), (src_b, dst_b, w_b))

        @pl.when(p == 1)
        def _():
            _process(s, (src_b, dst_b, w_b), (src_a, dst_a, w_a))
        return 0
    lax.fori_loop(0, SCH, sbody, 0)

    plsc.subcore_barrier()
    pltpu.sync_copy(acc.at[pl.ds(base, RPT)], parts_hbm.at[pl.ds(base, RPT)])


def _scb(xs, src, dst, w):
    return pl.kernel(
        _scb_body,
        out_type=jax.ShapeDtypeStruct((N_SP, H), jnp.float32),
        mesh=plsc.VectorSubcoreMesh(
            core_axis_name="c", subcore_axis_name="s",
            num_cores=1, num_subcores=NS),
        compiler_params=_SC_PARAMS,
        scratch_types=[
            pltpu.VMEM((ICH, K), jnp.int32),
            pltpu.VMEM((ICH, K), jnp.int32),
            pltpu.VMEM((ICH, K), jnp.int32),
            pltpu.VMEM((ICH, K), jnp.int32),
            pltpu.VMEM((ICH, K), jnp.float32),
            pltpu.VMEM((ICH, K), jnp.float32),
            pltpu.VMEM((K, H), jnp.float32),
            pltpu.VMEM((K, H), jnp.float32),
            pltpu.VMEM_SHARED((N_SP, H), jnp.float32),
            pltpu.SemaphoreType.DMA,
            pltpu.SemaphoreType.DMA,
            pltpu.SemaphoreType.DMA,
            pltpu.SemaphoreType.DMA,
            pltpu.SemaphoreType.DMA,
        ],
    )(xs, src, dst, w)


def _scc_body(pa_hbm, ps_hbm, row_hbm, col_hbm, out_hbm,
              pa_v, ps_v, row_v, col_v, o_v, lsem):
    wid = lax.axis_index("s") * NC + lax.axis_index("c")
    start = wid * DPT
    pltpu.async_copy(pa_hbm, pa_v, lsem)
    pltpu.async_copy(ps_hbm, ps_v, lsem)
    pltpu.async_copy(row_hbm.at[pl.ds(start, ELT)], row_v.at[pl.ds(0, ELT)],
                     lsem)
    pltpu.async_copy(col_hbm.at[pl.ds(start, ELT)], col_v.at[pl.ds(0, ELT)],
                     lsem)

    @pl.when(wid < NW - 1)
    def _():
        pltpu.async_copy(row_hbm.at[pl.ds(start + ELT, DPT - ELT)],
                         row_v.at[pl.ds(ELT, DPT - ELT)], lsem)
        pltpu.async_copy(col_hbm.at[pl.ds(start + ELT, DPT - ELT)],
                         col_v.at[pl.ds(ELT, DPT - ELT)], lsem)
        pltpu.make_async_copy(row_hbm.at[pl.ds(start + ELT, DPT - ELT)],
                              row_v.at[pl.ds(ELT, DPT - ELT)], lsem).wait()
        pltpu.make_async_copy(col_hbm.at[pl.ds(start + ELT, DPT - ELT)],
                              col_v.at[pl.ds(ELT, DPT - ELT)], lsem).wait()

    pltpu.make_async_copy(pa_hbm, pa_v, lsem).wait()
    pltpu.make_async_copy(ps_hbm, ps_v, lsem).wait()
    pltpu.make_async_copy(row_hbm.at[pl.ds(start, ELT)],
                          row_v.at[pl.ds(0, ELT)], lsem).wait()
    pltpu.make_async_copy(col_hbm.at[pl.ds(start, ELT)],
                          col_v.at[pl.ds(0, ELT)], lsem).wait()

    zeros16 = jnp.zeros((16,), jnp.int32)

    @pl.when(wid == NW - 1)
    def _():
        def tzero(i, _):
            sl = pl.ds(ELT + i * 16, 16)
            row_v[sl] = zeros16
            col_v[sl] = zeros16
            return 0
        lax.fori_loop(0, (DPT - ELT) // 16, tzero, 0)

    def gbody(g, _):
        sl = pl.ds(g * 16, 16)
        rv = row_v[sl]
        cv = col_v[sl]
        logit = plsc.load_gather(pa_v, [rv]) + plsc.load_gather(ps_v, [cv])
        o_v[sl] = 1.0 / (1.0 + jnp.exp(-logit))
        return 0
    lax.fori_loop(0, DPT // 16, gbody, 0)

    pltpu.sync_copy(o_v, out_hbm.at[wid])


def _scc(pa, ps, row, col):
    return pl.kernel(
        _scc_body,
        out_type=jax.ShapeDtypeStruct((NW, DPT), jnp.float32),
        mesh=_mesh(),
        compiler_params=_SC_PARAMS,
        scratch_types=[
            pltpu.VMEM((N_A,), jnp.float32),
            pltpu.VMEM((N_SP,), jnp.float32),
            pltpu.VMEM((DPT,), jnp.int32),
            pltpu.VMEM((DPT,), jnp.int32),
            pltpu.VMEM((DPT,), jnp.float32),
            pltpu.SemaphoreType.DMA,
        ],
    )(pa, ps, row, col)


def kernel(x_artwork, x_style, edge_index, edge_label_index,
           W_src, W_dst, att_src, att_dst, b_conv, W_head, b_head):
    src = edge_index[0].astype(jnp.int32)
    dst = edge_index[1].astype(jnp.int32)
    row = edge_label_index[0].astype(jnp.int32)
    col = edge_label_index[1].astype(jnp.int32)

    w1 = W_head[:H, 0]
    w2 = W_head[H:, 0]
    bh = jnp.broadcast_to(b_head, (N_SP,))

    xs, a_s, a_d, p_a = _tc1(x_artwork, x_style, W_src, W_dst,
                             att_src, att_dst, w1)

    src2 = src.reshape(NW, EPT)
    dst2 = dst.reshape(NW, EPT)
    w_e, dn = _sca(a_s, a_d, src2, dst2)

    parts = _scb(xs,
                 src.reshape(NS, SCH, ICH, K), dst.reshape(NS, SCH, ICH, K),
                 w_e.reshape(NS, SCH, ICH, K))

    ps = _tc2(parts, dn, b_conv, w2, bh)

    out = _scc(p_a, ps, row, col)

    return out.reshape(ELP)[:EL].reshape(EL, 1)


# speedup vs baseline: 26.0140x; 1.0053x over previous
import jax
import jax.numpy as jnp
from jax import lax
from jax.experimental import pallas as pl
from jax.experimental.pallas import tpu as pltpu
from jax.experimental.pallas import tpu_sc as plsc

N_A = 10000
N_S = 10000
E = 320000
EL = 100000
D = 128
H = 128

NC = 2
NS = 16
NW = NC * NS

EPT = E // NW
EPC = E // NS
K = 80
NCH = EPT // K
NCH2 = EPC // K
SCH = 25
ICH = NCH2 // SCH
SC_E = EPC // SCH
HH = H // 2

ELP = 100352
DPT = ELP // NW
ELT = EL - (NW - 1) * DPT
N_SP = 10240
RPT = N_SP // NS
ZR = 32

_HI = lax.Precision.HIGHEST


def _tc1_body(xa_ref, xst_ref, wsrc_ref, wdst_ref, asrc_ref, adst_ref, w1_ref,
              xs_ref, a_s_ref, a_d_ref, p_a_ref):
    xa = xa_ref[...]
    xs = jnp.dot(xa, wsrc_ref[...], precision=_HI)
    xs_ref[...] = xs
    a_s_ref[...] = jnp.sum(xs * asrc_ref[...][None, :], axis=1)
    v = jnp.sum(wdst_ref[...] * adst_ref[...][None, :], axis=1)
    a_d_ref[...] = jnp.sum(xst_ref[...] * v[None, :], axis=1)
    p_a_ref[...] = jnp.sum(xa * w1_ref[...][None, :], axis=1)


def _tc1(xa, xst, wsrc, wdst, asrc, adst, w1):
    return pl.pallas_call(
        _tc1_body,
        out_shape=[
            jax.ShapeDtypeStruct((N_A, H), jnp.float32),
            jax.ShapeDtypeStruct((N_A,), jnp.float32),
            jax.ShapeDtypeStruct((N_S,), jnp.float32),
            jax.ShapeDtypeStruct((N_A,), jnp.float32),
        ],
    )(xa, xst, wsrc, wdst, asrc, adst, w1)


def _tc2_body(parts_ref, dn_ref, bconv_ref, w2_ref, bh_ref, ps_ref):
    s = jnp.sum(dn_ref[...], axis=0)
    r = 1.0 / (s + 1e-16)
    z = jnp.maximum(parts_ref[...] * r[:, None] + bconv_ref[...][None, :], 0.0)
    ps_ref[...] = jnp.sum(z * w2_ref[...][None, :], axis=1) + bh_ref[...]


def _tc2(parts, dn, bconv, w2, bh):
    return pl.pallas_call(
        _tc2_body,
        out_shape=jax.ShapeDtypeStruct((N_SP,), jnp.float32),
    )(parts, dn, bconv, w2, bh)


def _mesh():
    return plsc.VectorSubcoreMesh(
        core_axis_name="c", subcore_axis_name="s",
        num_cores=NC, num_subcores=NS)


_SC_PARAMS = pltpu.CompilerParams(needs_layout_passes=False)


def _sca_body(a_s_hbm, a_d_hbm, src_hbm, dst_hbm, w_hbm, dn_hbm,
              as_v, ad_v, src_v, dst_v, w_v, dn_v, lsem):
    wid = lax.axis_index("s") * NC + lax.axis_index("c")
    pltpu.async_copy(a_s_hbm, as_v, lsem)
    pltpu.async_copy(a_d_hbm, ad_v, lsem)
    pltpu.async_copy(src_hbm.at[wid], src_v, lsem)
    pltpu.async_copy(dst_hbm.at[wid], dst_v, lsem)

    zeros = jnp.zeros((16,), jnp.float32)

    def zbody(i, _):
        dn_v[pl.ds(i * 16, 16)] = zeros
        return 0
    lax.fori_loop(0, N_SP // 16, zbody, 0)

    pltpu.make_async_copy(a_s_hbm, as_v, lsem).wait()
    pltpu.make_async_copy(a_d_hbm, ad_v, lsem).wait()
    pltpu.make_async_copy(src_hbm.at[wid], src_v, lsem).wait()
    pltpu.make_async_copy(dst_hbm.at[wid], dst_v, lsem).wait()

    def ebody(g, _):
        sl = pl.ds(g * 16, 16)
        sv = src_v[sl]
        dv = dst_v[sl]
        e = plsc.load_gather(as_v, [sv]) + plsc.load_gather(ad_v, [dv])
        e = jnp.where(e >= 0.0, e, 0.2 * e)
        w = jnp.exp(e)
        w_v[sl] = w
        plsc.addupdate_scatter(dn_v, [dv], w)
        return 0
    lax.fori_loop(0, EPT // 16, ebody, 0)

    pltpu.sync_copy(w_v, w_hbm.at[wid])
    pltpu.sync_copy(dn_v, dn_hbm.at[wid])


def _sca(a_s, a_d, src, dst):
    return pl.kernel(
        _sca_body,
        out_type=[
            jax.ShapeDtypeStruct((NW, EPT), jnp.float32),
            jax.ShapeDtypeStruct((NW, N_SP), jnp.float32),
        ],
        mesh=_mesh(),
        compiler_params=_SC_PARAMS,
        scratch_types=[
            pltpu.VMEM((N_A,), jnp.float32),
            pltpu.VMEM((N_S,), jnp.float32),
            pltpu.VMEM((EPT,), jnp.int32),
            pltpu.VMEM((EPT,), jnp.int32),
            pltpu.VMEM((EPT,), jnp.float32),
            pltpu.VMEM((N_SP,), jnp.float32),
            pltpu.SemaphoreType.DMA,
        ],
    )(a_s, a_d, src, dst)


def _scb_body(xs_hbm, src_hbm, dst_hbm, w_hbm, parts_hbm,
              src_a, src_b, dst_a, dst_b, w_a, w_b,
              rows_a, rows_b, acc, gsa, gsb, ssa, ssb, stsem):
    sid = lax.axis_index("s")

    zeros = jnp.zeros((16,), jnp.float32)

    def zbody(i, _):
        for h in range(H // 16):
            rows_a[i, pl.ds(h * 16, 16)] = zeros
        return 0
    lax.fori_loop(0, K, zbody, 0)
    base = sid * RPT
    for j in range(RPT // K):
        pltpu.sync_copy(rows_a, acc.at[pl.ds(base + j * K, K)])
    plsc.subcore_barrier()

    bufs = (rows_a, rows_b)
    gsems = (gsa, gsb)
    ssems = (ssa, ssb)

    def _stage(s, stg):
        src_s, dst_s, w_s = stg
        pltpu.async_copy(src_hbm.at[sid, s], src_s, stsem)
        pltpu.async_copy(dst_hbm.at[sid, s], dst_s, stsem)
        pltpu.async_copy(w_hbm.at[sid, s], w_s, stsem)

    def _stage_wait(s, stg):
        src_s, dst_s, w_s = stg
        pltpu.make_async_copy(src_hbm.at[sid, s], src_s, stsem).wait()
        pltpu.make_async_copy(dst_hbm.at[sid, s], dst_s, stsem).wait()
        pltpu.make_async_copy(w_hbm.at[sid, s], w_s, stsem).wait()

    _stage(0, (src_a, dst_a, w_a))

    def _process(s, stg, stg_next):
        src_s, dst_s, w_s = stg
        _stage_wait(s, stg)

        @pl.when(s + 1 < SCH)
        def _():
            _stage(s + 1, stg_next)

        pltpu.async_copy(xs_hbm.at[src_s.at[0]], rows_a, gsa)

        def ibody(i, _):
            for b in range(2):
                cc = 2 * i + b
                buf, oth = bufs[b], bufs[1 - b]
                pltpu.make_async_copy(
                    xs_hbm.at[src_s.at[cc]], buf, gsems[b]).wait()
                if b == 1:
                    pltpu.make_async_copy(
                        oth, acc.at[dst_s.at[cc - 1]], ssems[0]).wait()
                else:
                    @pl.when(i >= 1)
                    def _():
                        pltpu.make_async_copy(
                            oth, acc.at[dst_s.at[cc - 1]], ssems[1]).wait()
                if b == 0:
                    pltpu.async_copy(
                        xs_hbm.at[src_s.at[cc + 1]], oth, gsems[1])
                else:
                    @pl.when(i < ICH // 2 - 1)
                    def _():
                        pltpu.async_copy(
                            xs_hbm.at[src_s.at[cc + 1]], oth, gsems[0])

                c16 = jnp.full((16,), cc, jnp.int32)

                def rbody(r_g, _):
                    for u in range(4):
                        r_i = r_g * 4 + u
                        i16 = jnp.full((16,), r_i, jnp.int32)
                        cv = plsc.load_gather(w_s, [c16, i16])
                        for h in range(H // 16):
                            sl = pl.ds(h * 16, 16)
                            buf[r_i, sl] = buf[r_i, sl] * cv
                    return 0
                lax.fori_loop(0, K // 4, rbody, 0)

                pltpu.async_copy(buf, acc.at[dst_s.at[cc]],
                                 ssems[b], add=True)
            return 0
        lax.fori_loop(0, ICH // 2, ibody, 0)
        pltpu.make_async_copy(
            rows_b, acc.at[dst_s.at[ICH - 1]], ssb).wait()

    def sbody(s, _):
        p = s % 2

        @pl.when(p == 0)
        def _():
            _process(s, (src_a, dst_a, w_a), (src_b, dst_b, w_b))

        @pl.when(p == 1)
        def _():
            _process(s, (src_b, dst_b, w_b), (src_a, dst_a, w_a))
        return 0
    lax.fori_loop(0, SCH, sbody, 0)

    plsc.subcore_barrier()
    pltpu.sync_copy(acc.at[pl.ds(base, RPT)], parts_hbm.at[pl.ds(base, RPT)])


def _scb(xs, src, dst, w):
    return pl.kernel(
        _scb_body,
        out_type=jax.ShapeDtypeStruct((N_SP, H), jnp.float32),
        mesh=plsc.VectorSubcoreMesh(
            core_axis_name="c", subcore_axis_name="s",
            num_cores=1, num_subcores=NS),
        compiler_params=_SC_PARAMS,
        scratch_types=[
            pltpu.VMEM((ICH, K), jnp.int32),
            pltpu.VMEM((ICH, K), jnp.int32),
            pltpu.VMEM((ICH, K), jnp.int32),
            pltpu.VMEM((ICH, K), jnp.int32),
            pltpu.VMEM((ICH, K), jnp.float32),
            pltpu.VMEM((ICH, K), jnp.float32),
            pltpu.VMEM((K, H), jnp.float32),
            pltpu.VMEM((K, H), jnp.float32),
            pltpu.VMEM_SHARED((N_SP, H), jnp.float32),
            pltpu.SemaphoreType.DMA,
            pltpu.SemaphoreType.DMA,
            pltpu.SemaphoreType.DMA,
            pltpu.SemaphoreType.DMA,
            pltpu.SemaphoreType.DMA,
        ],
    )(xs, src, dst, w)


def _scc_body(pa_hbm, ps_hbm, row_hbm, col_hbm, out_hbm,
              pa_v, ps_v, row_v, col_v, o_v, lsem):
    wid = lax.axis_index("s") * NC + lax.axis_index("c")
    start = wid * DPT
    pltpu.async_copy(pa_hbm, pa_v, lsem)
    pltpu.async_copy(ps_hbm, ps_v, lsem)
    pltpu.async_copy(row_hbm.at[pl.ds(start, ELT)], row_v.at[pl.ds(0, ELT)],
                     lsem)
    pltpu.async_copy(col_hbm.at[pl.ds(start, ELT)], col_v.at[pl.ds(0, ELT)],
                     lsem)

    @pl.when(wid < NW - 1)
    def _():
        pltpu.async_copy(row_hbm.at[pl.ds(start + ELT, DPT - ELT)],
                         row_v.at[pl.ds(ELT, DPT - ELT)], lsem)
        pltpu.async_copy(col_hbm.at[pl.ds(start + ELT, DPT - ELT)],
                         col_v.at[pl.ds(ELT, DPT - ELT)], lsem)
        pltpu.make_async_copy(row_hbm.at[pl.ds(start + ELT, DPT - ELT)],
                              row_v.at[pl.ds(ELT, DPT - ELT)], lsem).wait()
        pltpu.make_async_copy(col_hbm.at[pl.ds(start + ELT, DPT - ELT)],
                              col_v.at[pl.ds(ELT, DPT - ELT)], lsem).wait()

    pltpu.make_async_copy(pa_hbm, pa_v, lsem).wait()
    pltpu.make_async_copy(ps_hbm, ps_v, lsem).wait()
    pltpu.make_async_copy(row_hbm.at[pl.ds(start, ELT)],
                          row_v.at[pl.ds(0, ELT)], lsem).wait()
    pltpu.make_async_copy(col_hbm.at[pl.ds(start, ELT)],
                          col_v.at[pl.ds(0, ELT)], lsem).wait()

    zeros16 = jnp.zeros((16,), jnp.int32)

    @pl.when(wid == NW - 1)
    def _():
        def tzero(i, _):
            sl = pl.ds(ELT + i * 16, 16)
            row_v[sl] = zeros16
            col_v[sl] = zeros16
            return 0
        lax.fori_loop(0, (DPT - ELT) // 16, tzero, 0)

    def gbody(g, _):
        sl = pl.ds(g * 16, 16)
        rv = row_v[sl]
        cv = col_v[sl]
        logit = plsc.load_gather(pa_v, [rv]) + plsc.load_gather(ps_v, [cv])
        o_v[sl] = 1.0 / (1.0 + jnp.exp(-logit))
        return 0
    lax.fori_loop(0, DPT // 16, gbody, 0)

    pltpu.sync_copy(o_v, out_hbm.at[wid])


def _scc(pa, ps, row, col):
    return pl.kernel(
        _scc_body,
        out_type=jax.ShapeDtypeStruct((NW, DPT), jnp.float32),
        mesh=_mesh(),
        compiler_params=_SC_PARAMS,
        scratch_types=[
            pltpu.VMEM((N_A,), jnp.float32),
            pltpu.VMEM((N_SP,), jnp.float32),
            pltpu.VMEM((DPT,), jnp.int32),
            pltpu.VMEM((DPT,), jnp.int32),
            pltpu.VMEM((DPT,), jnp.float32),
            pltpu.SemaphoreType.DMA,
        ],
    )(pa, ps, row, col)


def kernel(x_artwork, x_style, edge_index, edge_label_index,
           W_src, W_dst, att_src, att_dst, b_conv, W_head, b_head):
    src = edge_index[0].astype(jnp.int32)
    dst = edge_index[1].astype(jnp.int32)
    row = edge_label_index[0].astype(jnp.int32)
    col = edge_label_index[1].astype(jnp.int32)

    w1 = W_head[:H, 0]
    w2 = W_head[H:, 0]
    bh = jnp.broadcast_to(b_head, (N_SP,))

    xs, a_s, a_d, p_a = _tc1(x_artwork, x_style, W_src, W_dst,
                             att_src, att_dst, w1)

    src2 = src.reshape(NW, EPT)
    dst2 = dst.reshape(NW, EPT)
    w_e, dn = _sca(a_s, a_d, src2, dst2)

    parts = _scb(xs,
                 src.reshape(NS, SCH, ICH, K), dst.reshape(NS, SCH, ICH, K),
                 w_e.reshape(NS, SCH, ICH, K))

    ps = _tc2(parts, dn, b_conv, w2, bh)

    out = _scc(p_a, ps, row, col)

    return out.reshape(ELP)[:EL].reshape(EL, 1)


# speedup vs baseline: 26.0457x; 1.0012x over previous
import jax
import jax.numpy as jnp
from jax import lax
from jax.experimental import pallas as pl
from jax.experimental.pallas import tpu as pltpu
from jax.experimental.pallas import tpu_sc as plsc

N_A = 10000
N_S = 10000
E = 320000
EL = 100000
D = 128
H = 128

NC = 2
NS = 16
NW = NC * NS

EPT = E // NW
EPC = E // NS
K = 80
NCH = EPT // K
NCH2 = EPC // K
SCH = 25
ICH = NCH2 // SCH
SC_E = EPC // SCH
HH = H // 2

ELP = 100352
DPT = ELP // NW
ELT = EL - (NW - 1) * DPT
N_SP = 10240
RPT = N_SP // NS
ZR = 32

_HI = lax.Precision.HIGHEST


def _tc1_body(xa_ref, xst_ref, wsrc_ref, wdst_ref, asrc_ref, adst_ref, w1_ref,
              xs_ref, a_s_ref, a_d_ref, p_a_ref):
    xa = xa_ref[...]
    xs = jnp.dot(xa, wsrc_ref[...], precision=_HI)
    xs_ref[...] = xs
    a_s_ref[...] = jnp.sum(xs * asrc_ref[...][None, :], axis=1)
    v = jnp.sum(wdst_ref[...] * adst_ref[...][None, :], axis=1)
    a_d_ref[...] = jnp.sum(xst_ref[...] * v[None, :], axis=1)
    p_a_ref[...] = jnp.sum(xa * w1_ref[...][None, :], axis=1)


def _tc1(xa, xst, wsrc, wdst, asrc, adst, w1):
    return pl.pallas_call(
        _tc1_body,
        out_shape=[
            jax.ShapeDtypeStruct((N_A, H), jnp.float32),
            jax.ShapeDtypeStruct((N_A,), jnp.float32),
            jax.ShapeDtypeStruct((N_S,), jnp.float32),
            jax.ShapeDtypeStruct((N_A,), jnp.float32),
        ],
    )(xa, xst, wsrc, wdst, asrc, adst, w1)


def _tc2_body(parts_ref, dn_ref, bconv_ref, w2_ref, bh_ref, ps_ref):
    s = jnp.sum(dn_ref[...], axis=0)
    r = 1.0 / (s + 1e-16)
    z = jnp.maximum(parts_ref[...] * r[:, None] + bconv_ref[...][None, :], 0.0)
    ps_ref[...] = jnp.sum(z * w2_ref[...][None, :], axis=1) + bh_ref[...]


def _tc2(parts, dn, bconv, w2, bh):
    return pl.pallas_call(
        _tc2_body,
        out_shape=jax.ShapeDtypeStruct((N_SP,), jnp.float32),
    )(parts, dn, bconv, w2, bh)


def _mesh():
    return plsc.VectorSubcoreMesh(
        core_axis_name="c", subcore_axis_name="s",
        num_cores=NC, num_subcores=NS)


_SC_PARAMS = pltpu.CompilerParams(needs_layout_passes=False)


def _sca_body(a_s_hbm, a_d_hbm, src_hbm, dst_hbm, w_hbm, dn_hbm,
              as_v, ad_v, src_v, dst_v, w_v, dn_v, lsem):
    wid = lax.axis_index("s") * NC + lax.axis_index("c")
    pltpu.async_copy(a_s_hbm, as_v, lsem)
    pltpu.async_copy(a_d_hbm, ad_v, lsem)
    pltpu.async_copy(src_hbm.at[wid], src_v, lsem)
    pltpu.async_copy(dst_hbm.at[wid], dst_v, lsem)

    zeros = jnp.zeros((16,), jnp.float32)

    def zbody(i, _):
        dn_v[pl.ds(i * 16, 16)] = zeros
        return 0
    lax.fori_loop(0, N_SP // 16, zbody, 0)

    pltpu.make_async_copy(a_s_hbm, as_v, lsem).wait()
    pltpu.make_async_copy(a_d_hbm, ad_v, lsem).wait()
    pltpu.make_async_copy(src_hbm.at[wid], src_v, lsem).wait()
    pltpu.make_async_copy(dst_hbm.at[wid], dst_v, lsem).wait()

    def ebody(g, _):
        sl = pl.ds(g * 16, 16)
        sv = src_v[sl]
        dv = dst_v[sl]
        e = plsc.load_gather(as_v, [sv]) + plsc.load_gather(ad_v, [dv])
        e = jnp.where(e >= 0.0, e, 0.2 * e)
        w = jnp.exp(e)
        w_v[sl] = w
        plsc.addupdate_scatter(dn_v, [dv], w)
        return 0
    lax.fori_loop(0, EPT // 16, ebody, 0)

    pltpu.async_copy(w_v, w_hbm.at[wid], lsem)
    pltpu.async_copy(dn_v, dn_hbm.at[wid], lsem)
    pltpu.make_async_copy(w_v, w_hbm.at[wid], lsem).wait()
    pltpu.make_async_copy(dn_v, dn_hbm.at[wid], lsem).wait()


def _sca(a_s, a_d, src, dst):
    return pl.kernel(
        _sca_body,
        out_type=[
            jax.ShapeDtypeStruct((NW, EPT), jnp.float32),
            jax.ShapeDtypeStruct((NW, N_SP), jnp.float32),
        ],
        mesh=_mesh(),
        compiler_params=_SC_PARAMS,
        scratch_types=[
            pltpu.VMEM((N_A,), jnp.float32),
            pltpu.VMEM((N_S,), jnp.float32),
            pltpu.VMEM((EPT,), jnp.int32),
            pltpu.VMEM((EPT,), jnp.int32),
            pltpu.VMEM((EPT,), jnp.float32),
            pltpu.VMEM((N_SP,), jnp.float32),
            pltpu.SemaphoreType.DMA,
        ],
    )(a_s, a_d, src, dst)


def _scb_body(xs_hbm, src_hbm, dst_hbm, w_hbm, parts_hbm,
              src_a, src_b, dst_a, dst_b, w_a, w_b,
              rows_a, rows_b, acc, gsa, gsb, ssa, ssb, stsem):
    sid = lax.axis_index("s")

    zeros = jnp.zeros((16,), jnp.float32)

    def zbody(i, _):
        for h in range(H // 16):
            rows_a[i, pl.ds(h * 16, 16)] = zeros
        return 0
    lax.fori_loop(0, K, zbody, 0)
    base = sid * RPT
    for j in range(RPT // K):
        pltpu.async_copy(rows_a, acc.at[pl.ds(base + j * K, K)], gsa)
    for j in range(RPT // K):
        pltpu.make_async_copy(rows_a, acc.at[pl.ds(base + j * K, K)],
                              gsa).wait()
    plsc.subcore_barrier()

    bufs = (rows_a, rows_b)
    gsems = (gsa, gsb)
    ssems = (ssa, ssb)

    def _stage(s, stg):
        src_s, dst_s, w_s = stg
        pltpu.async_copy(src_hbm.at[sid, s], src_s, stsem)
        pltpu.async_copy(dst_hbm.at[sid, s], dst_s, stsem)
        pltpu.async_copy(w_hbm.at[sid, s], w_s, stsem)

    def _stage_wait(s, stg):
        src_s, dst_s, w_s = stg
        pltpu.make_async_copy(src_hbm.at[sid, s], src_s, stsem).wait()
        pltpu.make_async_copy(dst_hbm.at[sid, s], dst_s, stsem).wait()
        pltpu.make_async_copy(w_hbm.at[sid, s], w_s, stsem).wait()

    _stage(0, (src_a, dst_a, w_a))

    def _process(s, stg, stg_next):
        src_s, dst_s, w_s = stg
        _stage_wait(s, stg)

        @pl.when(s + 1 < SCH)
        def _():
            _stage(s + 1, stg_next)

        pltpu.async_copy(xs_hbm.at[src_s.at[0]], rows_a, gsa)

        def ibody(i, _):
            for b in range(2):
                cc = 2 * i + b
                buf, oth = bufs[b], bufs[1 - b]
                pltpu.make_async_copy(
                    xs_hbm.at[src_s.at[cc]], buf, gsems[b]).wait()
                if b == 1:
                    pltpu.make_async_copy(
                        oth, acc.at[dst_s.at[cc - 1]], ssems[0]).wait()
                else:
                    @pl.when(i >= 1)
                    def _():
                        pltpu.make_async_copy(
                            oth, acc.at[dst_s.at[cc - 1]], ssems[1]).wait()
                if b == 0:
                    pltpu.async_copy(
                        xs_hbm.at[src_s.at[cc + 1]], oth, gsems[1])
                else:
                    @pl.when(i < ICH // 2 - 1)
                    def _():
                        pltpu.async_copy(
                            xs_hbm.at[src_s.at[cc + 1]], oth, gsems[0])

                c16 = jnp.full((16,), cc, jnp.int32)

                def rbody(r_g, _):
                    for u in range(4):
                        r_i = r_g * 4 + u
                        i16 = jnp.full((16,), r_i, jnp.int32)
                        cv = plsc.load_gather(w_s, [c16, i16])
                        for h in range(H // 16):
                            sl = pl.ds(h * 16, 16)
                            buf[r_i, sl] = buf[r_i, sl] * cv
                    return 0
                lax.fori_loop(0, K // 4, rbody, 0)

                pltpu.async_copy(buf, acc.at[dst_s.at[cc]],
                                 ssems[b], add=True)
            return 0
        lax.fori_loop(0, ICH // 2, ibody, 0)
        pltpu.make_async_copy(
            rows_b, acc.at[dst_s.at[ICH - 1]], ssb).wait()

    def sbody(s, _):
        p = s % 2

        @pl.when(p == 0)
        def _():
            _process(s, (src_a, dst_a, w_a), (src_b, dst_b, w_b))

        @pl.when(p == 1)
        def _():
            _process(s, (src_b, dst_b, w_b), (src_a, dst_a, w_a))
        return 0
    lax.fori_loop(0, SCH, sbody, 0)

    plsc.subcore_barrier()
    pltpu.sync_copy(acc.at[pl.ds(base, RPT)], parts_hbm.at[pl.ds(base, RPT)])


def _scb(xs, src, dst, w):
    return pl.kernel(
        _scb_body,
        out_type=jax.ShapeDtypeStruct((N_SP, H), jnp.float32),
        mesh=plsc.VectorSubcoreMesh(
            core_axis_name="c", subcore_axis_name="s",
            num_cores=1, num_subcores=NS),
        compiler_params=_SC_PARAMS,
        scratch_types=[
            pltpu.VMEM((ICH, K), jnp.int32),
            pltpu.VMEM((ICH, K), jnp.int32),
            pltpu.VMEM((ICH, K), jnp.int32),
            pltpu.VMEM((ICH, K), jnp.int32),
            pltpu.VMEM((ICH, K), jnp.float32),
            pltpu.VMEM((ICH, K), jnp.float32),
            pltpu.VMEM((K, H), jnp.float32),
            pltpu.VMEM((K, H), jnp.float32),
            pltpu.VMEM_SHARED((N_SP, H), jnp.float32),
            pltpu.SemaphoreType.DMA,
            pltpu.SemaphoreType.DMA,
            pltpu.SemaphoreType.DMA,
            pltpu.SemaphoreType.DMA,
            pltpu.SemaphoreType.DMA,
        ],
    )(xs, src, dst, w)


def _scc_body(pa_hbm, ps_hbm, row_hbm, col_hbm, out_hbm,
              pa_v, ps_v, row_v, col_v, o_v, lsem):
    wid = lax.axis_index("s") * NC + lax.axis_index("c")
    start = wid * DPT
    pltpu.async_copy(pa_hbm, pa_v, lsem)
    pltpu.async_copy(ps_hbm, ps_v, lsem)
    pltpu.async_copy(row_hbm.at[pl.ds(start, ELT)], row_v.at[pl.ds(0, ELT)],
                     lsem)
    pltpu.async_copy(col_hbm.at[pl.ds(start, ELT)], col_v.at[pl.ds(0, ELT)],
                     lsem)

    @pl.when(wid < NW - 1)
    def _():
        pltpu.async_copy(row_hbm.at[pl.ds(start + ELT, DPT - ELT)],
                         row_v.at[pl.ds(ELT, DPT - ELT)], lsem)
        pltpu.async_copy(col_hbm.at[pl.ds(start + ELT, DPT - ELT)],
                         col_v.at[pl.ds(ELT, DPT - ELT)], lsem)
        pltpu.make_async_copy(row_hbm.at[pl.ds(start + ELT, DPT - ELT)],
                              row_v.at[pl.ds(ELT, DPT - ELT)], lsem).wait()
        pltpu.make_async_copy(col_hbm.at[pl.ds(start + ELT, DPT - ELT)],
                              col_v.at[pl.ds(ELT, DPT - ELT)], lsem).wait()

    pltpu.make_async_copy(pa_hbm, pa_v, lsem).wait()
    pltpu.make_async_copy(ps_hbm, ps_v, lsem).wait()
    pltpu.make_async_copy(row_hbm.at[pl.ds(start, ELT)],
                          row_v.at[pl.ds(0, ELT)], lsem).wait()
    pltpu.make_async_copy(col_hbm.at[pl.ds(start, ELT)],
                          col_v.at[pl.ds(0, ELT)], lsem).wait()

    zeros16 = jnp.zeros((16,), jnp.int32)

    @pl.when(wid == NW - 1)
    def _():
        def tzero(i, _):
            sl = pl.ds(ELT + i * 16, 16)
            row_v[sl] = zeros16
            col_v[sl] = zeros16
            return 0
        lax.fori_loop(0, (DPT - ELT) // 16, tzero, 0)

    def gbody(g, _):
        sl = pl.ds(g * 16, 16)
        rv = row_v[sl]
        cv = col_v[sl]
        logit = plsc.load_gather(pa_v, [rv]) + plsc.load_gather(ps_v, [cv])
        o_v[sl] = 1.0 / (1.0 + jnp.exp(-logit))
        return 0
    lax.fori_loop(0, DPT // 16, gbody, 0)

    pltpu.sync_copy(o_v, out_hbm.at[wid])


def _scc(pa, ps, row, col):
    return pl.kernel(
        _scc_body,
        out_type=jax.ShapeDtypeStruct((NW, DPT), jnp.float32),
        mesh=_mesh(),
        compiler_params=_SC_PARAMS,
        scratch_types=[
            pltpu.VMEM((N_A,), jnp.float32),
            pltpu.VMEM((N_SP,), jnp.float32),
            pltpu.VMEM((DPT,), jnp.int32),
            pltpu.VMEM((DPT,), jnp.int32),
            pltpu.VMEM((DPT,), jnp.float32),
            pltpu.SemaphoreType.DMA,
        ],
    )(pa, ps, row, col)


def kernel(x_artwork, x_style, edge_index, edge_label_index,
           W_src, W_dst, att_src, att_dst, b_conv, W_head, b_head):
    src = edge_index[0].astype(jnp.int32)
    dst = edge_index[1].astype(jnp.int32)
    row = edge_label_index[0].astype(jnp.int32)
    col = edge_label_index[1].astype(jnp.int32)

    w1 = W_head[:H, 0]
    w2 = W_head[H:, 0]
    bh = jnp.broadcast_to(b_head, (N_SP,))

    xs, a_s, a_d, p_a = _tc1(x_artwork, x_style, W_src, W_dst,
                             att_src, att_dst, w1)

    src2 = src.reshape(NW, EPT)
    dst2 = dst.reshape(NW, EPT)
    w_e, dn = _sca(a_s, a_d, src2, dst2)

    parts = _scb(xs,
                 src.reshape(NS, SCH, ICH, K), dst.reshape(NS, SCH, ICH, K),
                 w_e.reshape(NS, SCH, ICH, K))

    ps = _tc2(parts, dn, b_conv, w2, bh)

    out = _scc(p_a, ps, row, col)

    return out.reshape(ELP)[:EL].reshape(EL, 1)
